# baseline (speedup 1.0000x reference)
import functools

import jax
import jax.numpy as jnp
from jax import lax
from jax.experimental import pallas as pl
from jax.experimental.pallas import tpu as pltpu
from jax.experimental.pallas import tpu_sc as plsc

N0 = 10000
N1 = 160000
N2 = 50000
E0 = 320000
E1 = 320000
E2 = 150000
D = 128
HD = 64
H = 4
YW = 4 * HD + 16
ROWBLK = 400
C = 64
NT = 16
PADROW = N0
OUTPAD = 10240
A1PAD = N0 + 16
BLKCH = 32


def _blocks(e):
    return -(-e // (NT * C * BLKCH))

NB0 = _blocks(E0)
NB1 = _blocks(E1)
NB2 = _blocks(E2)


def _mm_t(x, w):
    return lax.dot_general(x, w, (((1,), (1,)), ((), ())),
                           preferred_element_type=jnp.float32)


def _mm(x, w):
    return lax.dot_general(x, w, (((1,), (0,)), ((), ())),
                           preferred_element_type=jnp.float32)


def _head_body(x_ref, w1_ref, b1_ref, wagg_ref, bagg_ref, ma1_ref, ma2_ref,
               brows_ref, y_ref, a1_ref, base_ref, *, blk, with_base):
    x = x_ref[...]
    a2acc = jnp.broadcast_to(brows_ref[1:2, :], (ROWBLK, 16))
    if with_base:
        a1acc = jnp.broadcast_to(brows_ref[0:1, :], (ROWBLK, 16))
        base = jnp.zeros((ROWBLK, D), jnp.float32)
    for h in range(H):
        xh = jax.nn.relu(_mm_t(x, w1_ref[h]) + b1_ref[h][None, :])
        yh = 0.25 * _mm_t(xh, wagg_ref[h, :, blk * D:(blk + 1) * D])
        y_ref[0, :, h * HD:(h + 1) * HD] = yh[:, :HD]
        y_ref[1, :, h * HD:(h + 1) * HD] = yh[:, HD:]
        a2acc = a2acc + _mm(xh, ma2_ref[h])
        if with_base:
            base = base + 0.25 * (_mm_t(xh, wagg_ref[h, :, 0:D])
                                  + bagg_ref[h][None, :])
            a1acc = a1acc + _mm(xh, ma1_ref[h])
    y_ref[0, :, 4 * HD:YW] = a2acc
    y_ref[1, :, 4 * HD:YW] = a2acc
    if with_base:
        a1_ref[...] = a1acc
        base_ref[...] = base


def _dense_call(x, w1, b1, wagg, bagg, ma1, ma2, brows, *, blk, with_base):
    n = x.shape[0]
    grid = (n // ROWBLK,)
    full = lambda shape: pl.BlockSpec(shape, lambda i: tuple(0 for _ in shape))
    in_specs = [
        pl.BlockSpec((ROWBLK, D), lambda i: (i, 0)),
        full((H, D, D)), full((H, D)), full((H, D, 4 * D)), full((H, D)),
        full((H, D, 16)), full((H, D, 16)), full((8, 16)),
    ]
    if with_base:
        out_shape = [
            jax.ShapeDtypeStruct((2, n, YW), jnp.float32),
            jax.ShapeDtypeStruct((n, 16), jnp.float32),
            jax.ShapeDtypeStruct((n, D), jnp.float32),
        ]
        out_specs = [
            pl.BlockSpec((2, ROWBLK, YW), lambda i: (0, i, 0)),
            pl.BlockSpec((ROWBLK, 16), lambda i: (i, 0)),
            pl.BlockSpec((ROWBLK, D), lambda i: (i, 0)),
        ]
        body = functools.partial(_head_body, blk=blk, with_base=True)
    else:
        out_shape = [jax.ShapeDtypeStruct((2, n, YW), jnp.float32)]
        out_specs = [pl.BlockSpec((2, ROWBLK, YW), lambda i: (0, i, 0))]

        def body(x_ref, w1_ref, b1_ref, wagg_ref, bagg_ref, ma1_ref, ma2_ref,
                 brows_ref, y_ref):
            _head_body(x_ref, w1_ref, b1_ref, wagg_ref, bagg_ref, ma1_ref,
                       ma2_ref, brows_ref, y_ref, None, None,
                       blk=blk, with_base=False)

    return pl.pallas_call(
        body, grid=grid, in_specs=in_specs, out_specs=out_specs,
        out_shape=out_shape,
    )(x, w1, b1, wagg, bagg, ma1, ma2, brows)


def _sc_edges(y0, y1, y2, a1t, i0, i1, i2):
    mesh = plsc.VectorSubcoreMesh(core_axis_name="c", subcore_axis_name="s")

    @functools.partial(
        pl.kernel, mesh=mesh,
        out_type=jax.ShapeDtypeStruct((2, OUTPAD, HD), jnp.float32),
        compiler_params=pltpu.CompilerParams(use_tc_tiling_on_sc=False),
        scratch_types=[
            pltpu.VMEM_SHARED((OUTPAD, HD), jnp.float32),
            pltpu.VMEM((32, HD), jnp.float32),
            pltpu.VMEM((2, BLKCH, 2, C), jnp.int32),
            pltpu.VMEM((2, C, 16), jnp.float32),
            pltpu.VMEM((2, C, YW), jnp.float32),
            pltpu.VMEM((2, C, HD), jnp.float32),
            pltpu.SemaphoreType.DMA,
            pltpu.SemaphoreType.DMA,
            pltpu.SemaphoreType.DMA,
            pltpu.SemaphoreType.DMA,
            pltpu.SemaphoreType.DMA,
        ],
    )
    def k(y0_hbm, y1_hbm, y2_hbm, a1_hbm, i0_hbm, i1_hbm, i2_hbm, out_hbm,
          acc_sp, zbuf, idxb, a1c, yc, vout, sem_i, sem_a0, sem_a1,
          sem_y0, sem_y1):
        cid = lax.axis_index("c")
        sid = lax.axis_index("s")
        sem_a = [sem_a0, sem_a1]
        sem_y = [sem_y0, sem_y1]

        def zrow(i, _):
            for j in range(HD // 16):
                zbuf[i, pl.ds(j * 16, 16)] = jnp.zeros((16,), jnp.float32)
            return 0
        lax.fori_loop(0, 32, zrow, 0)

        def zcp(i, _):
            pltpu.sync_copy(zbuf, acc_sp.at[pl.ds(sid * 640 + i * 32, 32)])
            return 0
        lax.fori_loop(0, 20, zcp, 0)
        plsc.subcore_barrier()

        def phase(i_hbm, y_hbm, nblk):
            nck = nblk * BLKCH

            def issue_gathers(c, slot):
                par = (c // BLKCH) % 2
                ent = c % BLKCH
                pltpu.async_copy(a1_hbm.at[idxb.at[par, ent, 0]],
                                 a1c.at[slot], sem_a[slot])
                pltpu.async_copy(y_hbm.at[cid].at[idxb.at[par, ent, 1]],
                                 yc.at[slot], sem_y[slot])

            def wait_gathers(slot):
                pltpu.make_async_copy(a1_hbm.at[pl.ds(0, C)],
                                      a1c.at[slot], sem_a[slot]).wait()
                pltpu.make_async_copy(y_hbm.at[0, pl.ds(0, C)],
                                      yc.at[slot], sem_y[slot]).wait()

            def compute(slot):
                def edge(e, _):
                    av = a1c[slot, e, :]
                    ys = yc[slot, e, pl.ds(4 * HD, 16)]
                    att = 1.0 / (1.0 + jnp.exp(-(av + ys)))
                    for j in range(HD // 16):
                        acc = None
                        for h in range(H):
                            s = att[h]
                            v = yc[slot, e, pl.ds(h * HD + j * 16, 16)]
                            acc = v * s if acc is None else acc + v * s
                        vout[slot, e, pl.ds(j * 16, 16)] = acc
                    return 0
                lax.fori_loop(0, C, edge, 0)

            pltpu.sync_copy(i_hbm.at[sid, 0], idxb.at[0])
            issue_gathers(0, 0)

            def pair(p, _):
                for kk in range(2):
                    c = 2 * p + kk
                    if kk == 0:
                        @pl.when(jnp.logical_and(c % BLKCH == 0,
                                                 c // BLKCH + 1 < nblk))
                        def _():
                            nb = c // BLKCH + 1
                            pltpu.async_copy(i_hbm.at[sid, nb],
                                             idxb.at[nb % 2], sem_i)
                    @pl.when(c + 1 < nck)
                    def _():
                        @pl.when((c + 1) % BLKCH == 0)
                        def _():
                            pltpu.make_async_copy(
                                i_hbm.at[sid, 0], idxb.at[0], sem_i).wait()
                        issue_gathers(c + 1, (kk + 1) % 2)
                    wait_gathers(kk)
                    compute(kk)
                    par = (c // BLKCH) % 2
                    ent = c % BLKCH
                    pltpu.sync_copy(vout.at[kk],
                                    acc_sp.at[idxb.at[par, ent, 0]],
                                    add=True)
                return 0
            lax.fori_loop(0, nck // 2, pair, 0)

        phase(i0_hbm, y0_hbm, NB0)
        phase(i1_hbm, y1_hbm, NB1)
        phase(i2_hbm, y2_hbm, NB2)
        plsc.subcore_barrier()

        def wcp(i, _):
            off = sid * 640 + i * 64
            pltpu.sync_copy(acc_sp.at[pl.ds(off, 64)],
                            out_hbm.at[cid, pl.ds(off, 64)])
            return 0
        lax.fori_loop(0, 10, wcp, 0)

    return k(y0, y1, y2, a1t, i0, i1, i2)


def kernel(x0_1, x1, x2, adj0_row, adj0_col, adj1_row, adj1_col,
           adj2_row, adj2_col, W1, b1, a1w, a1b, a2w, a2b, Wagg, bagg):
    f32 = jnp.float32
    i32 = jnp.int32

    ma1 = jnp.stack([jnp.zeros((D, 16), f32).at[:, h].set(a1w[h])
                     for h in range(H)])
    ma2 = jnp.stack([jnp.zeros((D, 16), f32).at[:, h].set(a2w[h])
                     for h in range(H)])
    brows = (jnp.zeros((8, 16), f32)
             .at[0, :H].set(a1b)
             .at[1, :H].set(a2b))

    y0, a1t, base = _dense_call(x0_1, W1, b1, Wagg, bagg, ma1, ma2, brows,
                                blk=1, with_base=True)
    (y1,) = _dense_call(x1, W1, b1, Wagg, bagg, ma1, ma2, brows,
                        blk=2, with_base=False)
    (y2,) = _dense_call(x2, W1, b1, Wagg, bagg, ma1, ma2, brows,
                        blk=3, with_base=False)

    a1p = jnp.concatenate([a1t, jnp.zeros((A1PAD - N0, 16), f32)])

    def pack_edges(rows, cols, nblk):
        e = rows.shape[0]
        ep = NT * C * BLKCH * nblk
        rp = jnp.concatenate([rows, jnp.full((ep - e,), PADROW, i32)])
        cp = jnp.concatenate([cols, jnp.zeros((ep - e,), i32)])
        rc = jnp.stack([rp.reshape(NT, nblk, BLKCH, C),
                        cp.reshape(NT, nblk, BLKCH, C)], axis=3)
        return rc

    i0 = pack_edges(adj0_row, adj0_col, NB0)
    i1 = pack_edges(adj1_row, adj1_col, NB1)
    i2 = pack_edges(adj2_row, adj2_col, NB2)

    outp = _sc_edges(y0, y1, y2, a1p, i0, i1, i2)
    return base + jnp.concatenate([outp[0, :N0], outp[1, :N0]], axis=1)


# speedup vs baseline: 1.1366x; 1.1366x over previous
import functools

import jax
import jax.numpy as jnp
from jax import lax
from jax.experimental import pallas as pl
from jax.experimental.pallas import tpu as pltpu
from jax.experimental.pallas import tpu_sc as plsc

N0 = 10000
N1 = 160000
N2 = 50000
E0 = 320000
E1 = 320000
E2 = 150000
D = 128
HD = 64
H = 4
YW = 4 * HD + 16
ROWBLK = 400
C = 64
NT = 16
PADROW = N0
OUTPAD = 10240
A1PAD = N0 + 16
BLKCH = 32


def _blocks(e):
    return -(-e // (NT * C * BLKCH))

NB0 = _blocks(E0)
NB1 = _blocks(E1)
NB2 = _blocks(E2)


def _mm_t(x, w):
    return lax.dot_general(x, w, (((1,), (1,)), ((), ())),
                           preferred_element_type=jnp.float32)


def _mm(x, w):
    return lax.dot_general(x, w, (((1,), (0,)), ((), ())),
                           preferred_element_type=jnp.float32)


def _head_body(x_ref, w1_ref, b1_ref, wagg_ref, bagg_ref, ma1_ref, ma2_ref,
               brows_ref, y_ref, a1_ref, base_ref, *, blk, with_base):
    x = x_ref[...]
    a2acc = jnp.broadcast_to(brows_ref[1:2, :], (ROWBLK, 16))
    if with_base:
        a1acc = jnp.broadcast_to(brows_ref[0:1, :], (ROWBLK, 16))
        base = jnp.zeros((ROWBLK, D), jnp.float32)
    for h in range(H):
        xh = jax.nn.relu(_mm_t(x, w1_ref[h]) + b1_ref[h][None, :])
        yh = 0.25 * _mm_t(xh, wagg_ref[h, :, blk * D:(blk + 1) * D])
        y_ref[0, :, h * HD:(h + 1) * HD] = yh[:, :HD]
        y_ref[1, :, h * HD:(h + 1) * HD] = yh[:, HD:]
        a2acc = a2acc + _mm(xh, ma2_ref[h])
        if with_base:
            base = base + 0.25 * (_mm_t(xh, wagg_ref[h, :, 0:D])
                                  + bagg_ref[h][None, :])
            a1acc = a1acc + _mm(xh, ma1_ref[h])
    y_ref[0, :, 4 * HD:YW] = a2acc
    y_ref[1, :, 4 * HD:YW] = a2acc
    if with_base:
        a1_ref[...] = a1acc
        base_ref[...] = base


def _dense_call(x, w1, b1, wagg, bagg, ma1, ma2, brows, *, blk, with_base):
    n = x.shape[0]
    grid = (n // ROWBLK,)
    full = lambda shape: pl.BlockSpec(shape, lambda i: tuple(0 for _ in shape))
    in_specs = [
        pl.BlockSpec((ROWBLK, D), lambda i: (i, 0)),
        full((H, D, D)), full((H, D)), full((H, D, 4 * D)), full((H, D)),
        full((H, D, 16)), full((H, D, 16)), full((8, 16)),
    ]
    if with_base:
        out_shape = [
            jax.ShapeDtypeStruct((2, n, YW), jnp.float32),
            jax.ShapeDtypeStruct((n, 16), jnp.float32),
            jax.ShapeDtypeStruct((n, D), jnp.float32),
        ]
        out_specs = [
            pl.BlockSpec((2, ROWBLK, YW), lambda i: (0, i, 0)),
            pl.BlockSpec((ROWBLK, 16), lambda i: (i, 0)),
            pl.BlockSpec((ROWBLK, D), lambda i: (i, 0)),
        ]
        body = functools.partial(_head_body, blk=blk, with_base=True)
    else:
        out_shape = [jax.ShapeDtypeStruct((2, n, YW), jnp.float32)]
        out_specs = [pl.BlockSpec((2, ROWBLK, YW), lambda i: (0, i, 0))]

        def body(x_ref, w1_ref, b1_ref, wagg_ref, bagg_ref, ma1_ref, ma2_ref,
                 brows_ref, y_ref):
            _head_body(x_ref, w1_ref, b1_ref, wagg_ref, bagg_ref, ma1_ref,
                       ma2_ref, brows_ref, y_ref, None, None,
                       blk=blk, with_base=False)

    return pl.pallas_call(
        body, grid=grid, in_specs=in_specs, out_specs=out_specs,
        out_shape=out_shape,
    )(x, w1, b1, wagg, bagg, ma1, ma2, brows)


def _sc_edges(y0, y1, y2, a1t, i0, i1, i2):
    mesh = plsc.VectorSubcoreMesh(core_axis_name="c", subcore_axis_name="s")

    @functools.partial(
        pl.kernel, mesh=mesh,
        out_type=jax.ShapeDtypeStruct((2, OUTPAD, HD), jnp.float32),
        compiler_params=pltpu.CompilerParams(use_tc_tiling_on_sc=False),
        scratch_types=[
            pltpu.VMEM_SHARED((OUTPAD, HD), jnp.float32),
            pltpu.VMEM((32, HD), jnp.float32),
            pltpu.VMEM((2, BLKCH, 2, C), jnp.int32),
            pltpu.VMEM((2, C, 16), jnp.float32),
            pltpu.VMEM((2, C, YW), jnp.float32),
            pltpu.VMEM((2, C, HD), jnp.float32),
            pltpu.SemaphoreType.DMA,
            pltpu.SemaphoreType.DMA,
            pltpu.SemaphoreType.DMA,
            pltpu.SemaphoreType.DMA,
            pltpu.SemaphoreType.DMA,
        ],
    )
    def k(y0_hbm, y1_hbm, y2_hbm, a1_hbm, i0_hbm, i1_hbm, i2_hbm, out_hbm,
          acc_sp, zbuf, idxb, a1c, yc, vout, sem_i, sem_a0, sem_a1,
          sem_y0, sem_y1):
        cid = lax.axis_index("c")
        sid = lax.axis_index("s")
        sem_a = [sem_a0, sem_a1]
        sem_y = [sem_y0, sem_y1]

        def zrow(i, _):
            for j in range(HD // 16):
                zbuf[i, pl.ds(j * 16, 16)] = jnp.zeros((16,), jnp.float32)
            return 0
        lax.fori_loop(0, 32, zrow, 0)

        def zcp(i, _):
            pltpu.sync_copy(zbuf, acc_sp.at[pl.ds(sid * 640 + i * 32, 32)])
            return 0
        lax.fori_loop(0, 20, zcp, 0)
        plsc.subcore_barrier()

        def phase(i_hbm, y_hbm, nblk):
            nck = nblk * BLKCH

            def issue_gathers(c, slot):
                par = (c // BLKCH) % 2
                ent = c % BLKCH
                pltpu.async_copy(a1_hbm.at[idxb.at[par, ent, 0]],
                                 a1c.at[slot], sem_a[slot])
                pltpu.async_copy(y_hbm.at[cid].at[idxb.at[par, ent, 1]],
                                 yc.at[slot], sem_y[slot])

            def wait_gathers(slot):
                pltpu.make_async_copy(a1_hbm.at[pl.ds(0, C)],
                                      a1c.at[slot], sem_a[slot]).wait()
                pltpu.make_async_copy(y_hbm.at[0, pl.ds(0, C)],
                                      yc.at[slot], sem_y[slot]).wait()

            def compute(slot):
                hidx = [jnp.full((16,), h, jnp.int32) for h in range(H)]

                def edge4(t, _):
                    for u in range(4):
                        e = t * 4 + u
                        av = a1c[slot, e, :]
                        ys = yc[slot, e, pl.ds(4 * HD, 16)]
                        att = 1.0 / (1.0 + jnp.exp(-(av + ys)))
                        acc = [None] * (HD // 16)
                        for h in range(H):
                            bh = jnp.take_along_axis(
                                att, hidx[h], axis=0,
                                mode="promise_in_bounds")
                            for j in range(HD // 16):
                                v = yc[slot, e, pl.ds(h * HD + j * 16, 16)]
                                acc[j] = (v * bh if acc[j] is None
                                          else acc[j] + v * bh)
                        for j in range(HD // 16):
                            vout[slot, e, pl.ds(j * 16, 16)] = acc[j]
                    return 0
                lax.fori_loop(0, C // 4, edge4, 0)

            pltpu.sync_copy(i_hbm.at[sid, 0], idxb.at[0])
            issue_gathers(0, 0)

            def pair(p, _):
                for kk in range(2):
                    c = 2 * p + kk
                    if kk == 0:
                        @pl.when(jnp.logical_and(c % BLKCH == 0,
                                                 c // BLKCH + 1 < nblk))
                        def _():
                            nb = c // BLKCH + 1
                            pltpu.async_copy(i_hbm.at[sid, nb],
                                             idxb.at[nb % 2], sem_i)
                    @pl.when(c + 1 < nck)
                    def _():
                        @pl.when((c + 1) % BLKCH == 0)
                        def _():
                            pltpu.make_async_copy(
                                i_hbm.at[sid, 0], idxb.at[0], sem_i).wait()
                        issue_gathers(c + 1, (kk + 1) % 2)
                    wait_gathers(kk)
                    compute(kk)
                    par = (c // BLKCH) % 2
                    ent = c % BLKCH
                    pltpu.sync_copy(vout.at[kk],
                                    acc_sp.at[idxb.at[par, ent, 0]],
                                    add=True)
                return 0
            lax.fori_loop(0, nck // 2, pair, 0)

        phase(i0_hbm, y0_hbm, NB0)
        phase(i1_hbm, y1_hbm, NB1)
        phase(i2_hbm, y2_hbm, NB2)
        plsc.subcore_barrier()

        def wcp(i, _):
            off = sid * 640 + i * 64
            pltpu.sync_copy(acc_sp.at[pl.ds(off, 64)],
                            out_hbm.at[cid, pl.ds(off, 64)])
            return 0
        lax.fori_loop(0, 10, wcp, 0)

    return k(y0, y1, y2, a1t, i0, i1, i2)


def kernel(x0_1, x1, x2, adj0_row, adj0_col, adj1_row, adj1_col,
           adj2_row, adj2_col, W1, b1, a1w, a1b, a2w, a2b, Wagg, bagg):
    f32 = jnp.float32
    i32 = jnp.int32

    ma1 = jnp.stack([jnp.zeros((D, 16), f32).at[:, h].set(a1w[h])
                     for h in range(H)])
    ma2 = jnp.stack([jnp.zeros((D, 16), f32).at[:, h].set(a2w[h])
                     for h in range(H)])
    brows = (jnp.zeros((8, 16), f32)
             .at[0, :H].set(a1b)
             .at[1, :H].set(a2b))

    y0, a1t, base = _dense_call(x0_1, W1, b1, Wagg, bagg, ma1, ma2, brows,
                                blk=1, with_base=True)
    (y1,) = _dense_call(x1, W1, b1, Wagg, bagg, ma1, ma2, brows,
                        blk=2, with_base=False)
    (y2,) = _dense_call(x2, W1, b1, Wagg, bagg, ma1, ma2, brows,
                        blk=3, with_base=False)

    a1p = jnp.concatenate([a1t, jnp.zeros((A1PAD - N0, 16), f32)])

    def pack_edges(rows, cols, nblk):
        e = rows.shape[0]
        ep = NT * C * BLKCH * nblk
        rp = jnp.concatenate([rows, jnp.full((ep - e,), PADROW, i32)])
        cp = jnp.concatenate([cols, jnp.zeros((ep - e,), i32)])
        rc = jnp.stack([rp.reshape(NT, nblk, BLKCH, C),
                        cp.reshape(NT, nblk, BLKCH, C)], axis=3)
        return rc

    i0 = pack_edges(adj0_row, adj0_col, NB0)
    i1 = pack_edges(adj1_row, adj1_col, NB1)
    i2 = pack_edges(adj2_row, adj2_col, NB2)

    outp = _sc_edges(y0, y1, y2, a1p, i0, i1, i2)
    return base + jnp.concatenate([outp[0, :N0], outp[1, :N0]], axis=1)


# speedup vs baseline: 1.6237x; 1.4286x over previous
import functools

import jax
import jax.numpy as jnp
from jax import lax
from jax.experimental import pallas as pl
from jax.experimental.pallas import tpu as pltpu
from jax.experimental.pallas import tpu_sc as plsc

N0 = 10000
N1 = 160000
N2 = 50000
E0 = 320000
E1 = 320000
E2 = 150000
D = 128
HD = 64
H = 4
YWI = 4 * (HD // 2) + 16
ROWBLK = 400
C = 128
NT = 16
PADROW = N0
OUTPAD = 10240
A1PAD = N0 + 16
BLKCH = 32


def _nck(e):
    n = -(-e // (NT * C))
    return n + (n % 2)

NCK0 = _nck(E0)
NCK1 = _nck(E1)
NCK2 = _nck(E2)
NB0 = -(-NCK0 // BLKCH)
NB1 = -(-NCK1 // BLKCH)
NB2 = -(-NCK2 // BLKCH)


def _mm_t(x, w):
    return lax.dot_general(x, w, (((1,), (1,)), ((), ())),
                           preferred_element_type=jnp.float32)


def _mm(x, w):
    return lax.dot_general(x, w, (((1,), (0,)), ((), ())),
                           preferred_element_type=jnp.float32)


def _bf16_bits(x):
    u = lax.bitcast_convert_type(x, jnp.int32)
    odd = lax.shift_right_logical(u, 16) & 1
    return lax.shift_right_logical(u + 0x7FFF + odd, 16)


def _head_body(x_ref, w1_ref, b1_ref, wagg_ref, bagg_ref, ma1_ref, ma2_ref,
               brows_ref, y_ref, a1_ref, base_ref, *, blk, with_base):
    x = x_ref[...]
    a2acc = jnp.broadcast_to(brows_ref[1:2, :], (ROWBLK, 16))
    if with_base:
        a1acc = jnp.broadcast_to(brows_ref[0:1, :], (ROWBLK, 16))
        base = jnp.zeros((ROWBLK, D), jnp.float32)
    for h in range(H):
        xh = jax.nn.relu(_mm_t(x, w1_ref[h]) + b1_ref[h][None, :])
        yh = 0.25 * _mm_t(xh, wagg_ref[h, :, blk * D:(blk + 1) * D])
        for p in range(2):
            lo = _bf16_bits(yh[:, p * HD:p * HD + 32])
            hi = _bf16_bits(yh[:, p * HD + 32:(p + 1) * HD])
            y_ref[p, :, h * 32:(h + 1) * 32] = lax.shift_left(hi, 16) | lo
        a2acc = a2acc + _mm(xh, ma2_ref[h])
        if with_base:
            base = base + 0.25 * (_mm_t(xh, wagg_ref[h, :, 0:D])
                                  + bagg_ref[h][None, :])
            a1acc = a1acc + _mm(xh, ma1_ref[h])
    tl = _bf16_bits(a2acc)
    y_ref[0, :, 4 * 32:YWI] = tl
    y_ref[1, :, 4 * 32:YWI] = tl
    if with_base:
        a1_ref[...] = a1acc
        base_ref[...] = base


def _dense_call(x, w1, b1, wagg, bagg, ma1, ma2, brows, *, blk, with_base):
    n = x.shape[0]
    grid = (n // ROWBLK,)
    full = lambda shape: pl.BlockSpec(shape, lambda i: tuple(0 for _ in shape))
    in_specs = [
        pl.BlockSpec((ROWBLK, D), lambda i: (i, 0)),
        full((H, D, D)), full((H, D)), full((H, D, 4 * D)), full((H, D)),
        full((H, D, 16)), full((H, D, 16)), full((8, 16)),
    ]
    if with_base:
        out_shape = [
            jax.ShapeDtypeStruct((2, n, YWI), jnp.int32),
            jax.ShapeDtypeStruct((n, 16), jnp.float32),
            jax.ShapeDtypeStruct((n, D), jnp.float32),
        ]
        out_specs = [
            pl.BlockSpec((2, ROWBLK, YWI), lambda i: (0, i, 0)),
            pl.BlockSpec((ROWBLK, 16), lambda i: (i, 0)),
            pl.BlockSpec((ROWBLK, D), lambda i: (i, 0)),
        ]
        body = functools.partial(_head_body, blk=blk, with_base=True)
    else:
        out_shape = [jax.ShapeDtypeStruct((2, n, YWI), jnp.int32)]
        out_specs = [pl.BlockSpec((2, ROWBLK, YWI), lambda i: (0, i, 0))]

        def body(x_ref, w1_ref, b1_ref, wagg_ref, bagg_ref, ma1_ref, ma2_ref,
                 brows_ref, y_ref):
            _head_body(x_ref, w1_ref, b1_ref, wagg_ref, bagg_ref, ma1_ref,
                       ma2_ref, brows_ref, y_ref, None, None,
                       blk=blk, with_base=False)

    return pl.pallas_call(
        body, grid=grid, in_specs=in_specs, out_specs=out_specs,
        out_shape=out_shape,
    )(x, w1, b1, wagg, bagg, ma1, ma2, brows)


def _sc_edges(y0, y1, y2, a1t, i0, i1, i2):
    mesh = plsc.VectorSubcoreMesh(core_axis_name="c", subcore_axis_name="s")

    @functools.partial(
        pl.kernel, mesh=mesh,
        out_type=jax.ShapeDtypeStruct((2, OUTPAD, HD), jnp.float32),
        compiler_params=pltpu.CompilerParams(use_tc_tiling_on_sc=False),
        scratch_types=[
            pltpu.VMEM_SHARED((OUTPAD, HD), jnp.float32),
            pltpu.VMEM_SHARED((A1PAD, 16), jnp.float32),
            pltpu.VMEM((32, HD), jnp.float32),
            pltpu.VMEM((2, BLKCH, 2, C), jnp.int32),
            pltpu.VMEM((2, C, 16), jnp.float32),
            pltpu.VMEM((2, C, YWI), jnp.int32),
            pltpu.VMEM((2, C, HD), jnp.float32),
            pltpu.SemaphoreType.DMA,
            pltpu.SemaphoreType.DMA,
            pltpu.SemaphoreType.DMA,
            pltpu.SemaphoreType.DMA,
            pltpu.SemaphoreType.DMA,
        ],
    )
    def k(y0_hbm, y1_hbm, y2_hbm, a1_hbm, i0_hbm, i1_hbm, i2_hbm, out_hbm,
          acc_sp, a1_sp, zbuf, idxb, a1c, yc, vout, sem_i, sem_a0, sem_a1,
          sem_y0, sem_y1):
        cid = lax.axis_index("c")
        sid = lax.axis_index("s")
        sem_a = [sem_a0, sem_a1]
        sem_y = [sem_y0, sem_y1]

        pltpu.sync_copy(a1_hbm.at[pl.ds(sid * (A1PAD // NT), A1PAD // NT)],
                        a1_sp.at[pl.ds(sid * (A1PAD // NT), A1PAD // NT)])

        def zrow(i, _):
            for j in range(HD // 16):
                zbuf[i, pl.ds(j * 16, 16)] = jnp.zeros((16,), jnp.float32)
            return 0
        lax.fori_loop(0, 32, zrow, 0)

        def zcp(i, _):
            pltpu.sync_copy(zbuf, acc_sp.at[pl.ds(sid * 640 + i * 32, 32)])
            return 0
        lax.fori_loop(0, 20, zcp, 0)
        plsc.subcore_barrier()

        def phase(i_hbm, y_hbm, nblk, nck):
            def issue_gathers(c, slot):
                par = (c // BLKCH) % 2
                ent = c % BLKCH
                pltpu.async_copy(a1_sp.at[idxb.at[par, ent, 0]],
                                 a1c.at[slot], sem_a[slot])
                pltpu.async_copy(y_hbm.at[cid].at[idxb.at[par, ent, 1]],
                                 yc.at[slot], sem_y[slot])

            def wait_gathers(slot):
                pltpu.make_async_copy(a1_hbm.at[pl.ds(0, C)],
                                      a1c.at[slot], sem_a[slot]).wait()
                pltpu.make_async_copy(y_hbm.at[0, pl.ds(0, C)],
                                      yc.at[slot], sem_y[slot]).wait()

            def compute(slot):
                hidx = [jnp.full((16,), h, jnp.int32) for h in range(H)]

                def unpk(w):
                    lo = lax.bitcast_convert_type(
                        lax.shift_left(w, 16), jnp.float32)
                    hi = lax.bitcast_convert_type(
                        w & jnp.int32(-65536), jnp.float32)
                    return lo, hi

                def edge4(t, _):
                    for u in range(4):
                        e = t * 4 + u
                        av = a1c[slot, e, :]
                        wt = yc[slot, e, pl.ds(4 * 32, 16)]
                        ys, _ = unpk(wt)
                        att = 1.0 / (1.0 + jnp.exp(-(av + ys)))
                        acc = [None] * (HD // 16)
                        for h in range(H):
                            bh = jnp.take_along_axis(
                                att, hidx[h], axis=0,
                                mode="promise_in_bounds")
                            for g in range(2):
                                w = yc[slot, e, pl.ds(h * 32 + g * 16, 16)]
                                f_lo, f_hi = unpk(w)
                                acc[g] = (f_lo * bh if acc[g] is None
                                          else acc[g] + f_lo * bh)
                                acc[g + 2] = (f_hi * bh if acc[g + 2] is None
                                              else acc[g + 2] + f_hi * bh)
                        for j in range(HD // 16):
                            vout[slot, e, pl.ds(j * 16, 16)] = acc[j]
                    return 0
                lax.fori_loop(0, C // 4, edge4, 0)

            pltpu.sync_copy(i_hbm.at[sid, 0], idxb.at[0])
            issue_gathers(0, 0)

            def pair(p, _):
                for kk in range(2):
                    c = 2 * p + kk
                    if kk == 0:
                        @pl.when(jnp.logical_and(c % BLKCH == 0,
                                                 c // BLKCH + 1 < nblk))
                        def _():
                            nb = c // BLKCH + 1
                            pltpu.async_copy(i_hbm.at[sid, nb],
                                             idxb.at[nb % 2], sem_i)
                    @pl.when(c + 1 < nck)
                    def _():
                        @pl.when((c + 1) % BLKCH == 0)
                        def _():
                            pltpu.make_async_copy(
                                i_hbm.at[sid, 0], idxb.at[0], sem_i).wait()
                        issue_gathers(c + 1, (kk + 1) % 2)
                    wait_gathers(kk)
                    compute(kk)
                    par = (c // BLKCH) % 2
                    ent = c % BLKCH
                    pltpu.sync_copy(vout.at[kk],
                                    acc_sp.at[idxb.at[par, ent, 0]],
                                    add=True)
                return 0
            lax.fori_loop(0, nck // 2, pair, 0)

        phase(i0_hbm, y0_hbm, NB0, NCK0)
        phase(i1_hbm, y1_hbm, NB1, NCK1)
        phase(i2_hbm, y2_hbm, NB2, NCK2)
        plsc.subcore_barrier()

        def wcp(i, _):
            off = sid * 640 + i * 64
            pltpu.sync_copy(acc_sp.at[pl.ds(off, 64)],
                            out_hbm.at[cid, pl.ds(off, 64)])
            return 0
        lax.fori_loop(0, 10, wcp, 0)

    return k(y0, y1, y2, a1t, i0, i1, i2)


def kernel(x0_1, x1, x2, adj0_row, adj0_col, adj1_row, adj1_col,
           adj2_row, adj2_col, W1, b1, a1w, a1b, a2w, a2b, Wagg, bagg):
    f32 = jnp.float32
    i32 = jnp.int32

    ma1 = jnp.stack([jnp.zeros((D, 16), f32).at[:, h].set(a1w[h])
                     for h in range(H)])
    ma2 = jnp.stack([jnp.zeros((D, 16), f32).at[:, h].set(a2w[h])
                     for h in range(H)])
    brows = (jnp.zeros((8, 16), f32)
             .at[0, :H].set(a1b)
             .at[1, :H].set(a2b))

    y0, a1t, base = _dense_call(x0_1, W1, b1, Wagg, bagg, ma1, ma2, brows,
                                blk=1, with_base=True)
    (y1,) = _dense_call(x1, W1, b1, Wagg, bagg, ma1, ma2, brows,
                        blk=2, with_base=False)
    (y2,) = _dense_call(x2, W1, b1, Wagg, bagg, ma1, ma2, brows,
                        blk=3, with_base=False)

    a1p = jnp.concatenate([a1t, jnp.zeros((A1PAD - N0, 16), f32)])

    def pack_edges(rows, cols, nblk, nck):
        e = rows.shape[0]
        ep = NT * C * nck
        slots = nblk * BLKCH
        rp = jnp.concatenate([rows, jnp.full((ep - e,), PADROW, i32)])
        cp = jnp.concatenate([cols, jnp.zeros((ep - e,), i32)])
        r3 = rp.reshape(NT, nck, C)
        c3 = cp.reshape(NT, nck, C)
        r3 = jnp.pad(r3, ((0, 0), (0, slots - nck), (0, 0)),
                     constant_values=PADROW)
        c3 = jnp.pad(c3, ((0, 0), (0, slots - nck), (0, 0)))
        rc = jnp.stack([r3.reshape(NT, nblk, BLKCH, C),
                        c3.reshape(NT, nblk, BLKCH, C)], axis=3)
        return rc

    i0 = pack_edges(adj0_row, adj0_col, NB0, NCK0)
    i1 = pack_edges(adj1_row, adj1_col, NB1, NCK1)
    i2 = pack_edges(adj2_row, adj2_col, NB2, NCK2)

    outp = _sc_edges(y0, y1, y2, a1p, i0, i1, i2)
    return base + jnp.concatenate([outp[0, :N0], outp[1, :N0]], axis=1)


# speedup vs baseline: 1.6394x; 1.0096x over previous
import functools

import jax
import jax.numpy as jnp
from jax import lax
from jax.experimental import pallas as pl
from jax.experimental.pallas import tpu as pltpu
from jax.experimental.pallas import tpu_sc as plsc

N0 = 10000
N1 = 160000
N2 = 50000
E0 = 320000
E1 = 320000
E2 = 150000
D = 128
HD = 64
H = 4
YWI = 4 * (HD // 2) + 16
ROWBLK = 400
C = 128
NT = 16
PADROW = N0
OUTPAD = 10240
A1PAD = N0 + 16
BLKCH = 32


def _nck(e):
    n = -(-e // (NT * C))
    return n + (n % 2)

NCK0 = _nck(E0)
NCK1 = _nck(E1)
NCK2 = _nck(E2)
NB0 = -(-NCK0 // BLKCH)
NB1 = -(-NCK1 // BLKCH)
NB2 = -(-NCK2 // BLKCH)


def _mm_t(x, w):
    return lax.dot_general(x, w, (((1,), (1,)), ((), ())),
                           preferred_element_type=jnp.float32)


def _mm(x, w):
    return lax.dot_general(x, w, (((1,), (0,)), ((), ())),
                           preferred_element_type=jnp.float32)


def _bf16_bits(x):
    u = lax.bitcast_convert_type(x, jnp.int32)
    odd = lax.shift_right_logical(u, 16) & 1
    return lax.shift_right_logical(u + 0x7FFF + odd, 16)


def _head_body(x_ref, w1_ref, b1_ref, wagg_ref, bagg_ref, ma1_ref, ma2_ref,
               brows_ref, y_ref, a1_ref, base_ref, *, blk, with_base):
    x = x_ref[...].astype(jnp.bfloat16)
    a2acc = jnp.broadcast_to(brows_ref[1:2, :], (ROWBLK, 16))
    if with_base:
        a1acc = jnp.broadcast_to(brows_ref[0:1, :], (ROWBLK, 16))
        base = jnp.zeros((ROWBLK, D), jnp.float32)
    for h in range(H):
        xh = jax.nn.relu(_mm_t(x, w1_ref[h]) + b1_ref[h][None, :])
        xhb = xh.astype(jnp.bfloat16)
        yh = 0.25 * _mm_t(xhb, wagg_ref[h, :, blk * D:(blk + 1) * D])
        for p in range(2):
            lo = _bf16_bits(yh[:, p * HD:p * HD + 32])
            hi = _bf16_bits(yh[:, p * HD + 32:(p + 1) * HD])
            y_ref[p, :, h * 32:(h + 1) * 32] = lax.shift_left(hi, 16) | lo
        a2acc = a2acc + _mm(xh, ma2_ref[h])
        if with_base:
            base = base + 0.25 * (_mm_t(xhb, wagg_ref[h, :, 0:D])
                                  + bagg_ref[h][None, :])
            a1acc = a1acc + _mm(xh, ma1_ref[h])
    tl = _bf16_bits(a2acc)
    y_ref[0, :, 4 * 32:YWI] = tl
    y_ref[1, :, 4 * 32:YWI] = tl
    if with_base:
        a1_ref[...] = a1acc
        base_ref[...] = base


def _dense_call(x, w1, b1, wagg, bagg, ma1, ma2, brows, *, blk, with_base):
    n = x.shape[0]
    grid = (n // ROWBLK,)
    full = lambda shape: pl.BlockSpec(shape, lambda i: tuple(0 for _ in shape))
    in_specs = [
        pl.BlockSpec((ROWBLK, D), lambda i: (i, 0)),
        full((H, D, D)), full((H, D)), full((H, D, 4 * D)), full((H, D)),
        full((H, D, 16)), full((H, D, 16)), full((8, 16)),
    ]
    if with_base:
        out_shape = [
            jax.ShapeDtypeStruct((2, n, YWI), jnp.int32),
            jax.ShapeDtypeStruct((n, 16), jnp.float32),
            jax.ShapeDtypeStruct((n, D), jnp.float32),
        ]
        out_specs = [
            pl.BlockSpec((2, ROWBLK, YWI), lambda i: (0, i, 0)),
            pl.BlockSpec((ROWBLK, 16), lambda i: (i, 0)),
            pl.BlockSpec((ROWBLK, D), lambda i: (i, 0)),
        ]
        body = functools.partial(_head_body, blk=blk, with_base=True)
    else:
        out_shape = [jax.ShapeDtypeStruct((2, n, YWI), jnp.int32)]
        out_specs = [pl.BlockSpec((2, ROWBLK, YWI), lambda i: (0, i, 0))]

        def body(x_ref, w1_ref, b1_ref, wagg_ref, bagg_ref, ma1_ref, ma2_ref,
                 brows_ref, y_ref):
            _head_body(x_ref, w1_ref, b1_ref, wagg_ref, bagg_ref, ma1_ref,
                       ma2_ref, brows_ref, y_ref, None, None,
                       blk=blk, with_base=False)

    return pl.pallas_call(
        body, grid=grid, in_specs=in_specs, out_specs=out_specs,
        out_shape=out_shape,
    )(x, w1, b1, wagg, bagg, ma1, ma2, brows)


def _sc_edges(y0, y1, y2, a1t, i0, i1, i2):
    mesh = plsc.VectorSubcoreMesh(core_axis_name="c", subcore_axis_name="s")

    @functools.partial(
        pl.kernel, mesh=mesh,
        out_type=jax.ShapeDtypeStruct((2, OUTPAD, HD), jnp.float32),
        compiler_params=pltpu.CompilerParams(use_tc_tiling_on_sc=False),
        scratch_types=[
            pltpu.VMEM_SHARED((OUTPAD, HD), jnp.float32),
            pltpu.VMEM_SHARED((A1PAD, 16), jnp.float32),
            pltpu.VMEM((32, HD), jnp.float32),
            pltpu.VMEM((2, BLKCH, 2, C), jnp.int32),
            pltpu.VMEM((2, C, 16), jnp.float32),
            pltpu.VMEM((2, C, YWI), jnp.int32),
            pltpu.VMEM((2, C, HD), jnp.float32),
            pltpu.SemaphoreType.DMA,
            pltpu.SemaphoreType.DMA,
            pltpu.SemaphoreType.DMA,
            pltpu.SemaphoreType.DMA,
            pltpu.SemaphoreType.DMA,
        ],
    )
    def k(y0_hbm, y1_hbm, y2_hbm, a1_hbm, i0_hbm, i1_hbm, i2_hbm, out_hbm,
          acc_sp, a1_sp, zbuf, idxb, a1c, yc, vout, sem_i, sem_a0, sem_a1,
          sem_y0, sem_y1):
        cid = lax.axis_index("c")
        sid = lax.axis_index("s")
        sem_a = [sem_a0, sem_a1]
        sem_y = [sem_y0, sem_y1]

        pltpu.sync_copy(a1_hbm.at[pl.ds(sid * (A1PAD // NT), A1PAD // NT)],
                        a1_sp.at[pl.ds(sid * (A1PAD // NT), A1PAD // NT)])

        def zrow(i, _):
            for j in range(HD // 16):
                zbuf[i, pl.ds(j * 16, 16)] = jnp.zeros((16,), jnp.float32)
            return 0
        lax.fori_loop(0, 32, zrow, 0)

        def zcp(i, _):
            pltpu.sync_copy(zbuf, acc_sp.at[pl.ds(sid * 640 + i * 32, 32)])
            return 0
        lax.fori_loop(0, 20, zcp, 0)
        plsc.subcore_barrier()

        def phase(i_hbm, y_hbm, nblk, nck):
            def issue_gathers(c, slot):
                par = (c // BLKCH) % 2
                ent = c % BLKCH
                pltpu.async_copy(a1_sp.at[idxb.at[par, ent, 0]],
                                 a1c.at[slot], sem_a[slot])
                pltpu.async_copy(y_hbm.at[cid].at[idxb.at[par, ent, 1]],
                                 yc.at[slot], sem_y[slot])

            def wait_gathers(slot):
                pltpu.make_async_copy(a1_hbm.at[pl.ds(0, C)],
                                      a1c.at[slot], sem_a[slot]).wait()
                pltpu.make_async_copy(y_hbm.at[0, pl.ds(0, C)],
                                      yc.at[slot], sem_y[slot]).wait()

            def compute(slot):
                hidx = [jnp.full((16,), h, jnp.int32) for h in range(H)]

                def unpk(w):
                    lo = lax.bitcast_convert_type(
                        lax.shift_left(w, 16), jnp.float32)
                    hi = lax.bitcast_convert_type(
                        w & jnp.int32(-65536), jnp.float32)
                    return lo, hi

                def edge4(t, _):
                    for u in range(4):
                        e = t * 4 + u
                        av = a1c[slot, e, :]
                        wt = yc[slot, e, pl.ds(4 * 32, 16)]
                        ys, _ = unpk(wt)
                        att = 1.0 / (1.0 + jnp.exp(-(av + ys)))
                        acc = [None] * (HD // 16)
                        for h in range(H):
                            bh = jnp.take_along_axis(
                                att, hidx[h], axis=0,
                                mode="promise_in_bounds")
                            for g in range(2):
                                w = yc[slot, e, pl.ds(h * 32 + g * 16, 16)]
                                f_lo, f_hi = unpk(w)
                                acc[g] = (f_lo * bh if acc[g] is None
                                          else acc[g] + f_lo * bh)
                                acc[g + 2] = (f_hi * bh if acc[g + 2] is None
                                              else acc[g + 2] + f_hi * bh)
                        for j in range(HD // 16):
                            vout[slot, e, pl.ds(j * 16, 16)] = acc[j]
                    return 0
                lax.fori_loop(0, C // 4, edge4, 0)

            pltpu.sync_copy(i_hbm.at[sid, 0], idxb.at[0])
            issue_gathers(0, 0)

            def pair(p, _):
                for kk in range(2):
                    c = 2 * p + kk
                    if kk == 0:
                        @pl.when(jnp.logical_and(c % BLKCH == 0,
                                                 c // BLKCH + 1 < nblk))
                        def _():
                            nb = c // BLKCH + 1
                            pltpu.async_copy(i_hbm.at[sid, nb],
                                             idxb.at[nb % 2], sem_i)
                    @pl.when(c + 1 < nck)
                    def _():
                        @pl.when((c + 1) % BLKCH == 0)
                        def _():
                            pltpu.make_async_copy(
                                i_hbm.at[sid, 0], idxb.at[0], sem_i).wait()
                        issue_gathers(c + 1, (kk + 1) % 2)
                    wait_gathers(kk)
                    compute(kk)
                    par = (c // BLKCH) % 2
                    ent = c % BLKCH
                    pltpu.sync_copy(vout.at[kk],
                                    acc_sp.at[idxb.at[par, ent, 0]],
                                    add=True)
                return 0
            lax.fori_loop(0, nck // 2, pair, 0)

        phase(i0_hbm, y0_hbm, NB0, NCK0)
        phase(i1_hbm, y1_hbm, NB1, NCK1)
        phase(i2_hbm, y2_hbm, NB2, NCK2)
        plsc.subcore_barrier()

        def wcp(i, _):
            off = sid * 640 + i * 64
            pltpu.sync_copy(acc_sp.at[pl.ds(off, 64)],
                            out_hbm.at[cid, pl.ds(off, 64)])
            return 0
        lax.fori_loop(0, 10, wcp, 0)

    return k(y0, y1, y2, a1t, i0, i1, i2)


def kernel(x0_1, x1, x2, adj0_row, adj0_col, adj1_row, adj1_col,
           adj2_row, adj2_col, W1, b1, a1w, a1b, a2w, a2b, Wagg, bagg):
    f32 = jnp.float32
    i32 = jnp.int32

    W1 = W1.astype(jnp.bfloat16)
    Wagg = Wagg.astype(jnp.bfloat16)
    ma1 = jnp.stack([jnp.zeros((D, 16), f32).at[:, h].set(a1w[h])
                     for h in range(H)])
    ma2 = jnp.stack([jnp.zeros((D, 16), f32).at[:, h].set(a2w[h])
                     for h in range(H)])
    brows = (jnp.zeros((8, 16), f32)
             .at[0, :H].set(a1b)
             .at[1, :H].set(a2b))

    y0, a1t, base = _dense_call(x0_1, W1, b1, Wagg, bagg, ma1, ma2, brows,
                                blk=1, with_base=True)
    (y1,) = _dense_call(x1, W1, b1, Wagg, bagg, ma1, ma2, brows,
                        blk=2, with_base=False)
    (y2,) = _dense_call(x2, W1, b1, Wagg, bagg, ma1, ma2, brows,
                        blk=3, with_base=False)

    a1p = jnp.concatenate([a1t, jnp.zeros((A1PAD - N0, 16), f32)])

    def pack_edges(rows, cols, nblk, nck):
        e = rows.shape[0]
        ep = NT * C * nck
        slots = nblk * BLKCH
        rp = jnp.concatenate([rows, jnp.full((ep - e,), PADROW, i32)])
        cp = jnp.concatenate([cols, jnp.zeros((ep - e,), i32)])
        r3 = rp.reshape(NT, nck, C)
        c3 = cp.reshape(NT, nck, C)
        r3 = jnp.pad(r3, ((0, 0), (0, slots - nck), (0, 0)),
                     constant_values=PADROW)
        c3 = jnp.pad(c3, ((0, 0), (0, slots - nck), (0, 0)))
        rc = jnp.stack([r3.reshape(NT, nblk, BLKCH, C),
                        c3.reshape(NT, nblk, BLKCH, C)], axis=3)
        return rc

    i0 = pack_edges(adj0_row, adj0_col, NB0, NCK0)
    i1 = pack_edges(adj1_row, adj1_col, NB1, NCK1)
    i2 = pack_edges(adj2_row, adj2_col, NB2, NCK2)

    outp = _sc_edges(y0, y1, y2, a1p, i0, i1, i2)
    return base + jnp.concatenate([outp[0, :N0], outp[1, :N0]], axis=1)


# speedup vs baseline: 1.7281x; 1.0541x over previous
import functools

import jax
import jax.numpy as jnp
from jax import lax
from jax.experimental import pallas as pl
from jax.experimental.pallas import tpu as pltpu
from jax.experimental.pallas import tpu_sc as plsc

N0 = 10000
N1 = 160000
N2 = 50000
E0 = 320000
E1 = 320000
E2 = 150000
D = 128
HD = 64
H = 4
YWI = 4 * (HD // 2) + 16
ROWBLK = 400
C = 128
NT = 16
PADROW = N0
OUTPAD = 10240
A1PAD = N0 + 16
BLKCH = 32


def _nck(e):
    n = -(-e // (NT * C))
    return n + (n % 2)

NCK0 = _nck(E0)
NCK1 = _nck(E1)
NCK2 = _nck(E2)
NB0 = -(-NCK0 // BLKCH)
NB1 = -(-NCK1 // BLKCH)
NB2 = -(-NCK2 // BLKCH)


def _mm_t(x, w):
    return lax.dot_general(x, w, (((1,), (1,)), ((), ())),
                           preferred_element_type=jnp.float32)


def _mm(x, w):
    return lax.dot_general(x, w, (((1,), (0,)), ((), ())),
                           preferred_element_type=jnp.float32)


def _bf16_bits(x):
    u = lax.bitcast_convert_type(x, jnp.int32)
    odd = lax.shift_right_logical(u, 16) & 1
    return lax.shift_right_logical(u + 0x7FFF + odd, 16)


def _head_body(x_ref, w1_ref, b1_ref, wagg_ref, bagg_ref, ma1_ref, ma2_ref,
               brows_ref, y_ref, a1_ref, base_ref, *, blk, with_base):
    x = x_ref[...].astype(jnp.bfloat16)
    a2acc = jnp.broadcast_to(brows_ref[1:2, :], (ROWBLK, 16))
    if with_base:
        a1acc = jnp.broadcast_to(brows_ref[0:1, :], (ROWBLK, 16))
        base = jnp.zeros((ROWBLK, D), jnp.float32)
    for h in range(H):
        xh = jax.nn.relu(_mm_t(x, w1_ref[h]) + b1_ref[h][None, :])
        xhb = xh.astype(jnp.bfloat16)
        yh = 0.25 * _mm_t(xhb, wagg_ref[h, :, blk * D:(blk + 1) * D])
        for p in range(2):
            lo = _bf16_bits(yh[:, p * HD:p * HD + 32])
            hi = _bf16_bits(yh[:, p * HD + 32:(p + 1) * HD])
            y_ref[p, :, h * 32:(h + 1) * 32] = lax.shift_left(hi, 16) | lo
        a2acc = a2acc + _mm(xh, ma2_ref[h])
        if with_base:
            base = base + 0.25 * (_mm_t(xhb, wagg_ref[h, :, 0:D])
                                  + bagg_ref[h][None, :])
            a1acc = a1acc + _mm(xh, ma1_ref[h])
    tl = _bf16_bits(a2acc)
    y_ref[0, :, 4 * 32:YWI] = tl
    y_ref[1, :, 4 * 32:YWI] = tl
    if with_base:
        a1_ref[...] = a1acc
        base_ref[...] = base


def _dense_call(x, w1, b1, wagg, bagg, ma1, ma2, brows, *, blk, with_base):
    n = x.shape[0]
    grid = (n // ROWBLK,)
    full = lambda shape: pl.BlockSpec(shape, lambda i: tuple(0 for _ in shape))
    in_specs = [
        pl.BlockSpec((ROWBLK, D), lambda i: (i, 0)),
        full((H, D, D)), full((H, D)), full((H, D, 4 * D)), full((H, D)),
        full((H, D, 16)), full((H, D, 16)), full((8, 16)),
    ]
    if with_base:
        out_shape = [
            jax.ShapeDtypeStruct((2, n, YWI), jnp.int32),
            jax.ShapeDtypeStruct((n, 16), jnp.float32),
            jax.ShapeDtypeStruct((n, D), jnp.float32),
        ]
        out_specs = [
            pl.BlockSpec((2, ROWBLK, YWI), lambda i: (0, i, 0)),
            pl.BlockSpec((ROWBLK, 16), lambda i: (i, 0)),
            pl.BlockSpec((ROWBLK, D), lambda i: (i, 0)),
        ]
        body = functools.partial(_head_body, blk=blk, with_base=True)
    else:
        out_shape = [jax.ShapeDtypeStruct((2, n, YWI), jnp.int32)]
        out_specs = [pl.BlockSpec((2, ROWBLK, YWI), lambda i: (0, i, 0))]

        def body(x_ref, w1_ref, b1_ref, wagg_ref, bagg_ref, ma1_ref, ma2_ref,
                 brows_ref, y_ref):
            _head_body(x_ref, w1_ref, b1_ref, wagg_ref, bagg_ref, ma1_ref,
                       ma2_ref, brows_ref, y_ref, None, None,
                       blk=blk, with_base=False)

    return pl.pallas_call(
        body, grid=grid, in_specs=in_specs, out_specs=out_specs,
        out_shape=out_shape,
    )(x, w1, b1, wagg, bagg, ma1, ma2, brows)


def _sc_edges(ys, a1t, idxs, nbs, ncks):
    mesh = plsc.VectorSubcoreMesh(core_axis_name="c", subcore_axis_name="s")
    nph = len(ys)

    @functools.partial(
        pl.kernel, mesh=mesh,
        out_type=jax.ShapeDtypeStruct((2, OUTPAD, HD), jnp.float32),
        compiler_params=pltpu.CompilerParams(use_tc_tiling_on_sc=False),
        scratch_types=[
            pltpu.VMEM_SHARED((OUTPAD, HD), jnp.float32),
            pltpu.VMEM_SHARED((A1PAD, 16), jnp.float32),
            pltpu.VMEM((32, HD), jnp.float32),
            pltpu.VMEM((2, BLKCH, 2, C), jnp.int32),
            pltpu.VMEM((2, C, 16), jnp.float32),
            pltpu.VMEM((2, C, YWI), jnp.int32),
            pltpu.VMEM((2, C, HD), jnp.float32),
            pltpu.SemaphoreType.DMA,
            pltpu.SemaphoreType.DMA,
            pltpu.SemaphoreType.DMA,
            pltpu.SemaphoreType.DMA,
            pltpu.SemaphoreType.DMA,
        ],
    )
    def k(*refs):
        y_hbms = refs[:nph]
        a1_hbm = refs[nph]
        i_hbms = refs[nph + 1:2 * nph + 1]
        (out_hbm, acc_sp, a1_sp, zbuf, idxb, a1c, yc, vout, sem_i,
         sem_a0, sem_a1, sem_y0, sem_y1) = refs[2 * nph + 1:]
        cid = lax.axis_index("c")
        sid = lax.axis_index("s")
        sem_a = [sem_a0, sem_a1]
        sem_y = [sem_y0, sem_y1]

        pltpu.sync_copy(a1_hbm.at[pl.ds(sid * (A1PAD // NT), A1PAD // NT)],
                        a1_sp.at[pl.ds(sid * (A1PAD // NT), A1PAD // NT)])

        def zrow(i, _):
            for j in range(HD // 16):
                zbuf[i, pl.ds(j * 16, 16)] = jnp.zeros((16,), jnp.float32)
            return 0
        lax.fori_loop(0, 32, zrow, 0)

        def zcp(i, _):
            pltpu.sync_copy(zbuf, acc_sp.at[pl.ds(sid * 640 + i * 32, 32)])
            return 0
        lax.fori_loop(0, 20, zcp, 0)
        plsc.subcore_barrier()

        def phase(i_hbm, y_hbm, nblk, nck):
            def issue_gathers(c, slot):
                par = (c // BLKCH) % 2
                ent = c % BLKCH
                pltpu.async_copy(a1_sp.at[idxb.at[par, ent, 0]],
                                 a1c.at[slot], sem_a[slot])
                pltpu.async_copy(y_hbm.at[cid].at[idxb.at[par, ent, 1]],
                                 yc.at[slot], sem_y[slot])

            def wait_gathers(slot):
                pltpu.make_async_copy(a1_hbm.at[pl.ds(0, C)],
                                      a1c.at[slot], sem_a[slot]).wait()
                pltpu.make_async_copy(y_hbm.at[0, pl.ds(0, C)],
                                      yc.at[slot], sem_y[slot]).wait()

            def compute(slot):
                hidx = [jnp.full((16,), h, jnp.int32) for h in range(H)]

                def unpk(w):
                    lo = lax.bitcast_convert_type(
                        lax.shift_left(w, 16), jnp.float32)
                    hi = lax.bitcast_convert_type(
                        w & jnp.int32(-65536), jnp.float32)
                    return lo, hi

                def edge4(t, _):
                    for u in range(4):
                        e = t * 4 + u
                        av = a1c[slot, e, :]
                        wt = yc[slot, e, pl.ds(4 * 32, 16)]
                        ys, _ = unpk(wt)
                        att = 1.0 / (1.0 + jnp.exp(-(av + ys)))
                        acc = [None] * (HD // 16)
                        for h in range(H):
                            bh = jnp.take_along_axis(
                                att, hidx[h], axis=0,
                                mode="promise_in_bounds")
                            for g in range(2):
                                w = yc[slot, e, pl.ds(h * 32 + g * 16, 16)]
                                f_lo, f_hi = unpk(w)
                                acc[g] = (f_lo * bh if acc[g] is None
                                          else acc[g] + f_lo * bh)
                                acc[g + 2] = (f_hi * bh if acc[g + 2] is None
                                              else acc[g + 2] + f_hi * bh)
                        for j in range(HD // 16):
                            vout[slot, e, pl.ds(j * 16, 16)] = acc[j]
                    return 0
                lax.fori_loop(0, C // 4, edge4, 0)

            pltpu.sync_copy(i_hbm.at[sid, 0], idxb.at[0])
            issue_gathers(0, 0)

            def pair(p, _):
                for kk in range(2):
                    c = 2 * p + kk
                    if kk == 0:
                        @pl.when(jnp.logical_and(c % BLKCH == 0,
                                                 c // BLKCH + 1 < nblk))
                        def _():
                            nb = c // BLKCH + 1
                            pltpu.async_copy(i_hbm.at[sid, nb],
                                             idxb.at[nb % 2], sem_i)
                    @pl.when(c + 1 < nck)
                    def _():
                        @pl.when((c + 1) % BLKCH == 0)
                        def _():
                            pltpu.make_async_copy(
                                i_hbm.at[sid, 0], idxb.at[0], sem_i).wait()
                        issue_gathers(c + 1, (kk + 1) % 2)
                    wait_gathers(kk)
                    compute(kk)
                    par = (c // BLKCH) % 2
                    ent = c % BLKCH
                    pltpu.sync_copy(vout.at[kk],
                                    acc_sp.at[idxb.at[par, ent, 0]],
                                    add=True)
                return 0
            lax.fori_loop(0, nck // 2, pair, 0)

        for ph in range(nph):
            phase(i_hbms[ph], y_hbms[ph], nbs[ph], ncks[ph])
        plsc.subcore_barrier()

        def wcp(i, _):
            off = sid * 640 + i * 64
            pltpu.sync_copy(acc_sp.at[pl.ds(off, 64)],
                            out_hbm.at[cid, pl.ds(off, 64)])
            return 0
        lax.fori_loop(0, 10, wcp, 0)

    return k(*ys, a1t, *idxs)


def kernel(x0_1, x1, x2, adj0_row, adj0_col, adj1_row, adj1_col,
           adj2_row, adj2_col, W1, b1, a1w, a1b, a2w, a2b, Wagg, bagg):
    f32 = jnp.float32
    i32 = jnp.int32

    W1 = W1.astype(jnp.bfloat16)
    Wagg = Wagg.astype(jnp.bfloat16)
    ma1 = jnp.stack([jnp.zeros((D, 16), f32).at[:, h].set(a1w[h])
                     for h in range(H)])
    ma2 = jnp.stack([jnp.zeros((D, 16), f32).at[:, h].set(a2w[h])
                     for h in range(H)])
    brows = (jnp.zeros((8, 16), f32)
             .at[0, :H].set(a1b)
             .at[1, :H].set(a2b))

    y0, a1t, base = _dense_call(x0_1, W1, b1, Wagg, bagg, ma1, ma2, brows,
                                blk=1, with_base=True)
    (y1,) = _dense_call(x1, W1, b1, Wagg, bagg, ma1, ma2, brows,
                        blk=2, with_base=False)

    a1p = jnp.concatenate([a1t, jnp.zeros((A1PAD - N0, 16), f32)])

    def pack_edges(rows, cols, nblk, nck):
        e = rows.shape[0]
        ep = NT * C * nck
        slots = nblk * BLKCH
        rp = jnp.concatenate([rows, jnp.full((ep - e,), PADROW, i32)])
        cp = jnp.concatenate([cols, jnp.zeros((ep - e,), i32)])
        r3 = rp.reshape(NT, nck, C)
        c3 = cp.reshape(NT, nck, C)
        r3 = jnp.pad(r3, ((0, 0), (0, slots - nck), (0, 0)),
                     constant_values=PADROW)
        c3 = jnp.pad(c3, ((0, 0), (0, slots - nck), (0, 0)))
        rc = jnp.stack([r3.reshape(NT, nblk, BLKCH, C),
                        c3.reshape(NT, nblk, BLKCH, C)], axis=3)
        return rc

    i0 = pack_edges(adj0_row, adj0_col, NB0, NCK0)
    i1 = pack_edges(adj1_row, adj1_col, NB1, NCK1)
    i2 = pack_edges(adj2_row, adj2_col, NB2, NCK2)

    outa = _sc_edges([y0, y1], a1p, [i0, i1], [NB0, NB1], [NCK0, NCK1])

    (y2,) = _dense_call(x2, W1, b1, Wagg, bagg, ma1, ma2, brows,
                        blk=3, with_base=False)
    outb = _sc_edges([y2], a1p, [i2], [NB2], [NCK2])

    outp = outa + outb
    return base + jnp.concatenate([outp[0, :N0], outp[1, :N0]], axis=1)


# speedup vs baseline: 1.8056x; 1.0449x over previous
import functools

import jax
import jax.numpy as jnp
from jax import lax
from jax.experimental import pallas as pl
from jax.experimental.pallas import tpu as pltpu
from jax.experimental.pallas import tpu_sc as plsc

N0 = 10000
N1 = 160000
N2 = 50000
E0 = 320000
E1 = 320000
E2 = 150000
D = 128
HD = 64
H = 4
YWI = 4 * (HD // 2) + 16
ROWBLK = 400
C = 128
NT = 16
PADROW = N0
OUTPAD = 10240
A1PAD = N0 + 16
BLKCH = 32


def _nck(e):
    n = -(-e // (NT * C))
    return n + (n % 2)

NCK0 = _nck(E0)
NCK1 = _nck(E1)
NCK2 = _nck(E2)
NB0 = -(-NCK0 // BLKCH)
NB1 = -(-NCK1 // BLKCH)
NB2 = -(-NCK2 // BLKCH)


def _mm_t(x, w):
    return lax.dot_general(x, w, (((1,), (1,)), ((), ())),
                           preferred_element_type=jnp.float32)


def _mm(x, w):
    return lax.dot_general(x, w, (((1,), (0,)), ((), ())),
                           preferred_element_type=jnp.float32)


def _bf16_bits(x):
    u = lax.bitcast_convert_type(x, jnp.int32)
    odd = lax.shift_right_logical(u, 16) & 1
    return lax.shift_right_logical(u + 0x7FFF + odd, 16)


def _head_body(x_ref, w1_ref, b1_ref, wagg_ref, bagg_ref, ma1_ref, ma2_ref,
               brows_ref, y_ref, a1_ref, base_ref, *, blk, with_base,
               nrows):
    x = x_ref[...].astype(jnp.bfloat16)
    a2acc = jnp.broadcast_to(brows_ref[1:2, :], (nrows, 16))
    if with_base:
        a1acc = jnp.broadcast_to(brows_ref[0:1, :], (nrows, 16))
        base = jnp.zeros((nrows, D), jnp.float32)
    for h in range(H):
        xh = jax.nn.relu(_mm_t(x, w1_ref[h]) + b1_ref[h][None, :])
        xhb = xh.astype(jnp.bfloat16)
        yh = 0.25 * _mm_t(xhb, wagg_ref[h, :, blk * D:(blk + 1) * D])
        for p in range(2):
            lo = _bf16_bits(yh[:, p * HD:p * HD + 32])
            hi = _bf16_bits(yh[:, p * HD + 32:(p + 1) * HD])
            y_ref[p, :, h * 32:(h + 1) * 32] = lax.shift_left(hi, 16) | lo
        a2acc = a2acc + _mm(xh, ma2_ref[h])
        if with_base:
            base = base + 0.25 * (_mm_t(xhb, wagg_ref[h, :, 0:D])
                                  + bagg_ref[h][None, :])
            a1acc = a1acc + _mm(xh, ma1_ref[h])
    tl = _bf16_bits(a2acc)
    y_ref[0, :, 4 * 32:YWI] = tl
    y_ref[1, :, 4 * 32:YWI] = tl
    if with_base:
        a1_ref[...] = a1acc
        base_ref[...] = base


def _dense_call(x, w1, b1, wagg, bagg, ma1, ma2, brows, *, blk, with_base,
                nrows=ROWBLK):
    n = x.shape[0]
    grid = (n // nrows,)
    full = lambda shape: pl.BlockSpec(shape, lambda i: tuple(0 for _ in shape))
    in_specs = [
        pl.BlockSpec((nrows, D), lambda i: (i, 0)),
        full((H, D, D)), full((H, D)), full((H, D, 4 * D)), full((H, D)),
        full((H, D, 16)), full((H, D, 16)), full((8, 16)),
    ]
    if with_base:
        out_shape = [
            jax.ShapeDtypeStruct((2, n, YWI), jnp.int32),
            jax.ShapeDtypeStruct((n, 16), jnp.float32),
            jax.ShapeDtypeStruct((n, D), jnp.float32),
        ]
        out_specs = [
            pl.BlockSpec((2, nrows, YWI), lambda i: (0, i, 0)),
            pl.BlockSpec((nrows, 16), lambda i: (i, 0)),
            pl.BlockSpec((nrows, D), lambda i: (i, 0)),
        ]
        body = functools.partial(_head_body, blk=blk, with_base=True,
                                 nrows=nrows)
    else:
        out_shape = [jax.ShapeDtypeStruct((2, n, YWI), jnp.int32)]
        out_specs = [pl.BlockSpec((2, nrows, YWI), lambda i: (0, i, 0))]

        def body(x_ref, w1_ref, b1_ref, wagg_ref, bagg_ref, ma1_ref, ma2_ref,
                 brows_ref, y_ref):
            _head_body(x_ref, w1_ref, b1_ref, wagg_ref, bagg_ref, ma1_ref,
                       ma2_ref, brows_ref, y_ref, None, None,
                       blk=blk, with_base=False, nrows=nrows)

    return pl.pallas_call(
        body, grid=grid, in_specs=in_specs, out_specs=out_specs,
        out_shape=out_shape,
    )(x, w1, b1, wagg, bagg, ma1, ma2, brows)


def _sc_edges(ys, a1t, idxs, nbs, ncks):
    mesh = plsc.VectorSubcoreMesh(core_axis_name="c", subcore_axis_name="s")
    nph = len(ys)

    @functools.partial(
        pl.kernel, mesh=mesh,
        out_type=jax.ShapeDtypeStruct((2, OUTPAD, HD), jnp.float32),
        compiler_params=pltpu.CompilerParams(use_tc_tiling_on_sc=False),
        scratch_types=[
            pltpu.VMEM_SHARED((OUTPAD, HD), jnp.float32),
            pltpu.VMEM_SHARED((A1PAD, 16), jnp.float32),
            pltpu.VMEM((32, HD), jnp.float32),
            pltpu.VMEM((2, BLKCH, 2, C), jnp.int32),
            pltpu.VMEM((2, C, 16), jnp.float32),
            pltpu.VMEM((2, C, YWI), jnp.int32),
            pltpu.VMEM((2, C, HD), jnp.float32),
            pltpu.SemaphoreType.DMA,
            pltpu.SemaphoreType.DMA,
            pltpu.SemaphoreType.DMA,
            pltpu.SemaphoreType.DMA,
            pltpu.SemaphoreType.DMA,
        ],
    )
    def k(*refs):
        y_hbms = refs[:nph]
        a1_hbm = refs[nph]
        i_hbms = refs[nph + 1:2 * nph + 1]
        (out_hbm, acc_sp, a1_sp, zbuf, idxb, a1c, yc, vout, sem_i,
         sem_a0, sem_a1, sem_y0, sem_y1) = refs[2 * nph + 1:]
        cid = lax.axis_index("c")
        sid = lax.axis_index("s")
        sem_a = [sem_a0, sem_a1]
        sem_y = [sem_y0, sem_y1]

        pltpu.sync_copy(a1_hbm.at[pl.ds(sid * (A1PAD // NT), A1PAD // NT)],
                        a1_sp.at[pl.ds(sid * (A1PAD // NT), A1PAD // NT)])

        def zrow(i, _):
            for j in range(HD // 16):
                zbuf[i, pl.ds(j * 16, 16)] = jnp.zeros((16,), jnp.float32)
            return 0
        lax.fori_loop(0, 32, zrow, 0)

        def zcp(i, _):
            pltpu.sync_copy(zbuf, acc_sp.at[pl.ds(sid * 640 + i * 32, 32)])
            return 0
        lax.fori_loop(0, 20, zcp, 0)
        plsc.subcore_barrier()

        def phase(i_hbm, y_hbm, nblk, nck):
            def issue_gathers(c, slot):
                par = (c // BLKCH) % 2
                ent = c % BLKCH
                pltpu.async_copy(a1_sp.at[idxb.at[par, ent, 0]],
                                 a1c.at[slot], sem_a[slot])
                pltpu.async_copy(y_hbm.at[cid].at[idxb.at[par, ent, 1]],
                                 yc.at[slot], sem_y[slot])

            def wait_gathers(slot):
                pltpu.make_async_copy(a1_hbm.at[pl.ds(0, C)],
                                      a1c.at[slot], sem_a[slot]).wait()
                pltpu.make_async_copy(y_hbm.at[0, pl.ds(0, C)],
                                      yc.at[slot], sem_y[slot]).wait()

            def compute(slot):
                hidx = [jnp.full((16,), h, jnp.int32) for h in range(H)]

                def unpk(w):
                    lo = lax.bitcast_convert_type(
                        lax.shift_left(w, 16), jnp.float32)
                    hi = lax.bitcast_convert_type(
                        w & jnp.int32(-65536), jnp.float32)
                    return lo, hi

                def edge4(t, _):
                    for u in range(4):
                        e = t * 4 + u
                        av = a1c[slot, e, :]
                        wt = yc[slot, e, pl.ds(4 * 32, 16)]
                        ys, _ = unpk(wt)
                        att = 1.0 / (1.0 + jnp.exp(-(av + ys)))
                        acc = [None] * (HD // 16)
                        for h in range(H):
                            bh = jnp.take_along_axis(
                                att, hidx[h], axis=0,
                                mode="promise_in_bounds")
                            for g in range(2):
                                w = yc[slot, e, pl.ds(h * 32 + g * 16, 16)]
                                f_lo, f_hi = unpk(w)
                                acc[g] = (f_lo * bh if acc[g] is None
                                          else acc[g] + f_lo * bh)
                                acc[g + 2] = (f_hi * bh if acc[g + 2] is None
                                              else acc[g + 2] + f_hi * bh)
                        for j in range(HD // 16):
                            vout[slot, e, pl.ds(j * 16, 16)] = acc[j]
                    return 0
                lax.fori_loop(0, C // 4, edge4, 0)

            pltpu.sync_copy(i_hbm.at[sid, 0], idxb.at[0])
            issue_gathers(0, 0)

            def pair(p, _):
                for kk in range(2):
                    c = 2 * p + kk
                    if kk == 0:
                        @pl.when(jnp.logical_and(c % BLKCH == 0,
                                                 c // BLKCH + 1 < nblk))
                        def _():
                            nb = c // BLKCH + 1
                            pltpu.async_copy(i_hbm.at[sid, nb],
                                             idxb.at[nb % 2], sem_i)
                    @pl.when(c + 1 < nck)
                    def _():
                        @pl.when((c + 1) % BLKCH == 0)
                        def _():
                            pltpu.make_async_copy(
                                i_hbm.at[sid, 0], idxb.at[0], sem_i).wait()
                        issue_gathers(c + 1, (kk + 1) % 2)
                    wait_gathers(kk)
                    compute(kk)
                    par = (c // BLKCH) % 2
                    ent = c % BLKCH
                    pltpu.sync_copy(vout.at[kk],
                                    acc_sp.at[idxb.at[par, ent, 0]],
                                    add=True)
                return 0
            lax.fori_loop(0, nck // 2, pair, 0)

        for ph in range(nph):
            phase(i_hbms[ph], y_hbms[ph], nbs[ph], ncks[ph])
        plsc.subcore_barrier()

        def wcp(i, _):
            off = sid * 640 + i * 64
            pltpu.sync_copy(acc_sp.at[pl.ds(off, 64)],
                            out_hbm.at[cid, pl.ds(off, 64)])
            return 0
        lax.fori_loop(0, 10, wcp, 0)

    return k(*ys, a1t, *idxs)


def kernel(x0_1, x1, x2, adj0_row, adj0_col, adj1_row, adj1_col,
           adj2_row, adj2_col, W1, b1, a1w, a1b, a2w, a2b, Wagg, bagg):
    f32 = jnp.float32
    i32 = jnp.int32

    W1 = W1.astype(jnp.bfloat16)
    Wagg = Wagg.astype(jnp.bfloat16)
    ma1 = jnp.stack([jnp.zeros((D, 16), f32).at[:, h].set(a1w[h])
                     for h in range(H)])
    ma2 = jnp.stack([jnp.zeros((D, 16), f32).at[:, h].set(a2w[h])
                     for h in range(H)])
    brows = (jnp.zeros((8, 16), f32)
             .at[0, :H].set(a1b)
             .at[1, :H].set(a2b))

    y0, a1t, base = _dense_call(x0_1, W1, b1, Wagg, bagg, ma1, ma2, brows,
                                blk=1, with_base=True)
    (y1,) = _dense_call(x1, W1, b1, Wagg, bagg, ma1, ma2, brows,
                        blk=2, with_base=False, nrows=1000)

    a1p = jnp.concatenate([a1t, jnp.zeros((A1PAD - N0, 16), f32)])

    def pack_edges(rows, cols, nblk, nck):
        e = rows.shape[0]
        ep = NT * C * nck
        slots = nblk * BLKCH
        rp = jnp.concatenate([rows, jnp.full((ep - e,), PADROW, i32)])
        cp = jnp.concatenate([cols, jnp.zeros((ep - e,), i32)])
        r3 = rp.reshape(NT, nck, C)
        c3 = cp.reshape(NT, nck, C)
        r3 = jnp.pad(r3, ((0, 0), (0, slots - nck), (0, 0)),
                     constant_values=PADROW)
        c3 = jnp.pad(c3, ((0, 0), (0, slots - nck), (0, 0)))
        rc = jnp.stack([r3.reshape(NT, nblk, BLKCH, C),
                        c3.reshape(NT, nblk, BLKCH, C)], axis=3)
        return rc

    i0 = pack_edges(adj0_row, adj0_col, NB0, NCK0)
    i1 = pack_edges(adj1_row, adj1_col, NB1, NCK1)
    i2 = pack_edges(adj2_row, adj2_col, NB2, NCK2)

    outa = _sc_edges([y0, y1], a1p, [i0, i1], [NB0, NB1], [NCK0, NCK1])

    (y2,) = _dense_call(x2, W1, b1, Wagg, bagg, ma1, ma2, brows,
                        blk=3, with_base=False, nrows=1000)
    outb = _sc_edges([y2], a1p, [i2], [NB2], [NCK2])

    outp = outa + outb
    return base + jnp.concatenate([outp[0, :N0], outp[1, :N0]], axis=1)


# speedup vs baseline: 1.8400x; 1.0190x over previous
import functools

import jax
import jax.numpy as jnp
from jax import lax
from jax.experimental import pallas as pl
from jax.experimental.pallas import tpu as pltpu
from jax.experimental.pallas import tpu_sc as plsc

N0 = 10000
N1 = 160000
N2 = 50000
E0 = 320000
E1 = 320000
E2 = 150000
D = 128
HD = 64
H = 4
YWI = 4 * (HD // 2) + 16
ROWBLK = 1000
C = 128
NT = 16
PADROW = N0
OUTPAD = 10240
A1PAD = N0 + 16
BLKCH = 32


def _nck(e):
    n = -(-e // (NT * C))
    return n + (n % 2)

NCK0 = _nck(E0)
NCK1 = _nck(E1)
NCK2 = _nck(E2)
NB0 = -(-NCK0 // BLKCH)
NB1 = -(-NCK1 // BLKCH)
NB2 = -(-NCK2 // BLKCH)


def _mm_t(x, w):
    return lax.dot_general(x, w, (((1,), (1,)), ((), ())),
                           preferred_element_type=jnp.float32)


def _mm(x, w):
    return lax.dot_general(x, w, (((1,), (0,)), ((), ())),
                           preferred_element_type=jnp.float32)


def _bf16_bits(x):
    u = lax.bitcast_convert_type(x, jnp.int32)
    odd = lax.shift_right_logical(u, 16) & 1
    return lax.shift_right_logical(u + 0x7FFF + odd, 16)


def _head_body(x_ref, w1_ref, b1_ref, wagg_ref, bagg_ref, ma1_ref, ma2_ref,
               brows_ref, y_ref, a1_ref, base_ref, *, blk, with_base,
               nrows):
    x = x_ref[...].astype(jnp.bfloat16)
    a2acc = jnp.broadcast_to(brows_ref[1:2, :], (nrows, 16))
    if with_base:
        a1acc = jnp.broadcast_to(brows_ref[0:1, :], (nrows, 16))
        base = jnp.zeros((nrows, D), jnp.float32)
    for h in range(H):
        xh = jax.nn.relu(_mm_t(x, w1_ref[h]) + b1_ref[h][None, :])
        xhb = xh.astype(jnp.bfloat16)
        yh = 0.25 * _mm_t(xhb, wagg_ref[h, :, blk * D:(blk + 1) * D])
        for p in range(2):
            lo = _bf16_bits(yh[:, p * HD:p * HD + 32])
            hi = _bf16_bits(yh[:, p * HD + 32:(p + 1) * HD])
            y_ref[p, :, h * 32:(h + 1) * 32] = lax.shift_left(hi, 16) | lo
        a2acc = a2acc + _mm(xh, ma2_ref[h])
        if with_base:
            base = base + 0.25 * (_mm_t(xhb, wagg_ref[h, :, 0:D])
                                  + bagg_ref[h][None, :])
            a1acc = a1acc + _mm(xh, ma1_ref[h])
    tl = _bf16_bits(a2acc)
    y_ref[0, :, 4 * 32:YWI] = tl
    y_ref[1, :, 4 * 32:YWI] = tl
    if with_base:
        a1_ref[...] = a1acc
        base_ref[...] = base


def _dense_call(x, w1, b1, wagg, bagg, ma1, ma2, brows, *, blk, with_base,
                nrows=ROWBLK):
    n = x.shape[0]
    grid = (n // nrows,)
    full = lambda shape: pl.BlockSpec(shape, lambda i: tuple(0 for _ in shape))
    in_specs = [
        pl.BlockSpec((nrows, D), lambda i: (i, 0)),
        full((H, D, D)), full((H, D)), full((H, D, 4 * D)), full((H, D)),
        full((H, D, 16)), full((H, D, 16)), full((8, 16)),
    ]
    if with_base:
        out_shape = [
            jax.ShapeDtypeStruct((2, n, YWI), jnp.int32),
            jax.ShapeDtypeStruct((n, 16), jnp.float32),
            jax.ShapeDtypeStruct((n, D), jnp.float32),
        ]
        out_specs = [
            pl.BlockSpec((2, nrows, YWI), lambda i: (0, i, 0)),
            pl.BlockSpec((nrows, 16), lambda i: (i, 0)),
            pl.BlockSpec((nrows, D), lambda i: (i, 0)),
        ]
        body = functools.partial(_head_body, blk=blk, with_base=True,
                                 nrows=nrows)
    else:
        out_shape = [jax.ShapeDtypeStruct((2, n, YWI), jnp.int32)]
        out_specs = [pl.BlockSpec((2, nrows, YWI), lambda i: (0, i, 0))]

        def body(x_ref, w1_ref, b1_ref, wagg_ref, bagg_ref, ma1_ref, ma2_ref,
                 brows_ref, y_ref):
            _head_body(x_ref, w1_ref, b1_ref, wagg_ref, bagg_ref, ma1_ref,
                       ma2_ref, brows_ref, y_ref, None, None,
                       blk=blk, with_base=False, nrows=nrows)

    return pl.pallas_call(
        body, grid=grid, in_specs=in_specs, out_specs=out_specs,
        out_shape=out_shape,
    )(x, w1, b1, wagg, bagg, ma1, ma2, brows)


def _sc_edges(ys, a1t, idxs, nbs, ncks):
    mesh = plsc.VectorSubcoreMesh(core_axis_name="c", subcore_axis_name="s")
    nph = len(ys)

    @functools.partial(
        pl.kernel, mesh=mesh,
        out_type=jax.ShapeDtypeStruct((2, OUTPAD, HD), jnp.float32),
        compiler_params=pltpu.CompilerParams(use_tc_tiling_on_sc=False),
        scratch_types=[
            pltpu.VMEM_SHARED((OUTPAD, HD), jnp.float32),
            pltpu.VMEM_SHARED((A1PAD, 16), jnp.float32),
            pltpu.VMEM((32, HD), jnp.float32),
            pltpu.VMEM((2, BLKCH, 2, C), jnp.int32),
            pltpu.VMEM((2, C, 16), jnp.float32),
            pltpu.VMEM((2, C, YWI), jnp.int32),
            pltpu.VMEM((2, C, HD), jnp.float32),
            pltpu.SemaphoreType.DMA,
            pltpu.SemaphoreType.DMA,
            pltpu.SemaphoreType.DMA,
            pltpu.SemaphoreType.DMA,
            pltpu.SemaphoreType.DMA,
        ],
    )
    def k(*refs):
        y_hbms = refs[:nph]
        a1_hbm = refs[nph]
        i_hbms = refs[nph + 1:2 * nph + 1]
        (out_hbm, acc_sp, a1_sp, zbuf, idxb, a1c, yc, vout, sem_i,
         sem_a0, sem_a1, sem_y0, sem_y1) = refs[2 * nph + 1:]
        cid = lax.axis_index("c")
        sid = lax.axis_index("s")
        sem_a = [sem_a0, sem_a1]
        sem_y = [sem_y0, sem_y1]

        pltpu.sync_copy(a1_hbm.at[pl.ds(sid * (A1PAD // NT), A1PAD // NT)],
                        a1_sp.at[pl.ds(sid * (A1PAD // NT), A1PAD // NT)])

        def zrow(i, _):
            for j in range(HD // 16):
                zbuf[i, pl.ds(j * 16, 16)] = jnp.zeros((16,), jnp.float32)
            return 0
        lax.fori_loop(0, 32, zrow, 0)

        def zcp(i, _):
            pltpu.sync_copy(zbuf, acc_sp.at[pl.ds(sid * 640 + i * 32, 32)])
            return 0
        lax.fori_loop(0, 20, zcp, 0)
        plsc.subcore_barrier()

        def phase(i_hbm, y_hbm, nblk, nck):
            def issue_gathers(c, slot):
                par = (c // BLKCH) % 2
                ent = c % BLKCH
                pltpu.async_copy(a1_sp.at[idxb.at[par, ent, 0]],
                                 a1c.at[slot], sem_a[slot])
                pltpu.async_copy(y_hbm.at[cid].at[idxb.at[par, ent, 1]],
                                 yc.at[slot], sem_y[slot])

            def wait_gathers(slot):
                pltpu.make_async_copy(a1_hbm.at[pl.ds(0, C)],
                                      a1c.at[slot], sem_a[slot]).wait()
                pltpu.make_async_copy(y_hbm.at[0, pl.ds(0, C)],
                                      yc.at[slot], sem_y[slot]).wait()

            def compute(slot):
                hidx = [jnp.full((16,), h, jnp.int32) for h in range(H)]

                def unpk(w):
                    lo = lax.bitcast_convert_type(
                        lax.shift_left(w, 16), jnp.float32)
                    hi = lax.bitcast_convert_type(
                        w & jnp.int32(-65536), jnp.float32)
                    return lo, hi

                def edge4(t, _):
                    for u in range(4):
                        e = t * 4 + u
                        av = a1c[slot, e, :]
                        wt = yc[slot, e, pl.ds(4 * 32, 16)]
                        ys, _ = unpk(wt)
                        att = 1.0 / (1.0 + jnp.exp(-(av + ys)))
                        acc = [None] * (HD // 16)
                        for h in range(H):
                            bh = jnp.take_along_axis(
                                att, hidx[h], axis=0,
                                mode="promise_in_bounds")
                            for g in range(2):
                                w = yc[slot, e, pl.ds(h * 32 + g * 16, 16)]
                                f_lo, f_hi = unpk(w)
                                acc[g] = (f_lo * bh if acc[g] is None
                                          else acc[g] + f_lo * bh)
                                acc[g + 2] = (f_hi * bh if acc[g + 2] is None
                                              else acc[g + 2] + f_hi * bh)
                        for j in range(HD // 16):
                            vout[slot, e, pl.ds(j * 16, 16)] = acc[j]
                    return 0
                lax.fori_loop(0, C // 4, edge4, 0)

            pltpu.sync_copy(i_hbm.at[sid, 0], idxb.at[0])
            issue_gathers(0, 0)

            def pair(p, _):
                for kk in range(2):
                    c = 2 * p + kk
                    if kk == 0:
                        @pl.when(jnp.logical_and(c % BLKCH == 0,
                                                 c // BLKCH + 1 < nblk))
                        def _():
                            nb = c // BLKCH + 1
                            pltpu.async_copy(i_hbm.at[sid, nb],
                                             idxb.at[nb % 2], sem_i)
                    @pl.when(c + 1 < nck)
                    def _():
                        @pl.when((c + 1) % BLKCH == 0)
                        def _():
                            pltpu.make_async_copy(
                                i_hbm.at[sid, 0], idxb.at[0], sem_i).wait()
                        issue_gathers(c + 1, (kk + 1) % 2)
                    wait_gathers(kk)
                    compute(kk)
                    par = (c // BLKCH) % 2
                    ent = c % BLKCH
                    pltpu.sync_copy(vout.at[kk],
                                    acc_sp.at[idxb.at[par, ent, 0]],
                                    add=True)
                return 0
            lax.fori_loop(0, nck // 2, pair, 0)

        for ph in range(nph):
            phase(i_hbms[ph], y_hbms[ph], nbs[ph], ncks[ph])
        plsc.subcore_barrier()

        def wcp(i, _):
            off = sid * 640 + i * 64
            pltpu.sync_copy(acc_sp.at[pl.ds(off, 64)],
                            out_hbm.at[cid, pl.ds(off, 64)])
            return 0
        lax.fori_loop(0, 10, wcp, 0)

    return k(*ys, a1t, *idxs)


def kernel(x0_1, x1, x2, adj0_row, adj0_col, adj1_row, adj1_col,
           adj2_row, adj2_col, W1, b1, a1w, a1b, a2w, a2b, Wagg, bagg):
    f32 = jnp.float32
    i32 = jnp.int32

    W1 = W1.astype(jnp.bfloat16)
    Wagg = Wagg.astype(jnp.bfloat16)
    ma1 = jnp.stack([jnp.zeros((D, 16), f32).at[:, h].set(a1w[h])
                     for h in range(H)])
    ma2 = jnp.stack([jnp.zeros((D, 16), f32).at[:, h].set(a2w[h])
                     for h in range(H)])
    brows = (jnp.zeros((8, 16), f32)
             .at[0, :H].set(a1b)
             .at[1, :H].set(a2b))

    y0, a1t, base = _dense_call(x0_1, W1, b1, Wagg, bagg, ma1, ma2, brows,
                                blk=1, with_base=True)
    (y1,) = _dense_call(x1, W1, b1, Wagg, bagg, ma1, ma2, brows,
                        blk=2, with_base=False, nrows=2000)

    a1p = jnp.concatenate([a1t, jnp.zeros((A1PAD - N0, 16), f32)])

    def pack_edges(rows, cols, nblk, nck):
        e = rows.shape[0]
        ep = NT * C * nck
        slots = nblk * BLKCH
        rp = jnp.concatenate([rows, jnp.full((ep - e,), PADROW, i32)])
        cp = jnp.concatenate([cols, jnp.zeros((ep - e,), i32)])
        r3 = rp.reshape(NT, nck, C)
        c3 = cp.reshape(NT, nck, C)
        r3 = jnp.pad(r3, ((0, 0), (0, slots - nck), (0, 0)),
                     constant_values=PADROW)
        c3 = jnp.pad(c3, ((0, 0), (0, slots - nck), (0, 0)))
        rc = jnp.stack([r3.reshape(NT, nblk, BLKCH, C),
                        c3.reshape(NT, nblk, BLKCH, C)], axis=3)
        return rc

    i0 = pack_edges(adj0_row, adj0_col, NB0, NCK0)
    i1 = pack_edges(adj1_row, adj1_col, NB1, NCK1)
    i2 = pack_edges(adj2_row, adj2_col, NB2, NCK2)

    outa = _sc_edges([y0, y1], a1p, [i0, i1], [NB0, NB1], [NCK0, NCK1])

    (y2,) = _dense_call(x2, W1, b1, Wagg, bagg, ma1, ma2, brows,
                        blk=3, with_base=False, nrows=2000)
    outb = _sc_edges([y2], a1p, [i2], [NB2], [NCK2])

    outp = outa + outb
    return base + jnp.concatenate([outp[0, :N0], outp[1, :N0]], axis=1)


# speedup vs baseline: 2.1139x; 1.1488x over previous
import functools

import jax
import jax.numpy as jnp
from jax import lax
from jax.experimental import pallas as pl
from jax.experimental.pallas import tpu as pltpu
from jax.experimental.pallas import tpu_sc as plsc

N0 = 10000
N1 = 160000
N2 = 50000
E0 = 320000
E1 = 320000
E2 = 150000
D = 128
HD = 64
H = 4
YWI = 4 * (HD // 2) + 16
ROWBLK = 1000
C = 128
NT = 16
PADROW = N0
OUTPAD = 10240
A1PAD = N0 + 16
BLKCH = 32


def _nck(e):
    n = -(-e // (NT * C))
    return n + (n % 2)

NCK0 = _nck(E0)
NCK1 = _nck(E1)
NCK2 = _nck(E2)
NB0 = -(-NCK0 // BLKCH)
NB1 = -(-NCK1 // BLKCH)
NB2 = -(-NCK2 // BLKCH)


def _mm_t(x, w):
    return lax.dot_general(x, w, (((1,), (1,)), ((), ())),
                           preferred_element_type=jnp.float32)


def _mm(x, w):
    return lax.dot_general(x, w, (((1,), (0,)), ((), ())),
                           preferred_element_type=jnp.float32)


def _bf16_bits(x):
    u = lax.bitcast_convert_type(x, jnp.int32)
    odd = lax.shift_right_logical(u, 16) & 1
    return lax.shift_right_logical(u + 0x7FFF + odd, 16)


def _head_body(x_ref, w1_ref, b1_ref, wagg_ref, bagg_ref, ma1_ref, ma2_ref,
               brows_ref, y_ref, a1_ref, base_ref, *, blk, with_base,
               nrows):
    x = x_ref[...].astype(jnp.bfloat16)
    a2acc = jnp.broadcast_to(brows_ref[1:2, :], (nrows, 16))
    if with_base:
        a1acc = jnp.broadcast_to(brows_ref[0:1, :], (nrows, 16))
        base = jnp.zeros((nrows, D), jnp.float32)
    for h in range(H):
        xh = jax.nn.relu(_mm_t(x, w1_ref[h]) + b1_ref[h][None, :])
        xhb = xh.astype(jnp.bfloat16)
        yh = 0.25 * _mm_t(xhb, wagg_ref[h, :, blk * D:(blk + 1) * D])
        for p in range(2):
            lo = _bf16_bits(yh[:, p * HD:p * HD + 32])
            hi = _bf16_bits(yh[:, p * HD + 32:(p + 1) * HD])
            y_ref[p, :, h * 32:(h + 1) * 32] = lax.shift_left(hi, 16) | lo
        a2acc = a2acc + _mm(xh, ma2_ref[h])
        if with_base:
            base = base + 0.25 * (_mm_t(xhb, wagg_ref[h, :, 0:D])
                                  + bagg_ref[h][None, :])
            a1acc = a1acc + _mm(xh, ma1_ref[h])
    tl = _bf16_bits(a2acc)
    y_ref[0, :, 4 * 32:YWI] = tl
    y_ref[1, :, 4 * 32:YWI] = tl
    if with_base:
        a1_ref[...] = a1acc
        base_ref[...] = base


def _dense_call(x, w1, b1, wagg, bagg, ma1, ma2, brows, *, blk, with_base,
                nrows=ROWBLK):
    n = x.shape[0]
    grid = (n // nrows,)
    full = lambda shape: pl.BlockSpec(shape, lambda i: tuple(0 for _ in shape))
    in_specs = [
        pl.BlockSpec((nrows, D), lambda i: (i, 0)),
        full((H, D, D)), full((H, D)), full((H, D, 4 * D)), full((H, D)),
        full((H, D, 16)), full((H, D, 16)), full((8, 16)),
    ]
    if with_base:
        out_shape = [
            jax.ShapeDtypeStruct((2, n, YWI), jnp.int32),
            jax.ShapeDtypeStruct((n, 16), jnp.float32),
            jax.ShapeDtypeStruct((n, D), jnp.float32),
        ]
        out_specs = [
            pl.BlockSpec((2, nrows, YWI), lambda i: (0, i, 0)),
            pl.BlockSpec((nrows, 16), lambda i: (i, 0)),
            pl.BlockSpec((nrows, D), lambda i: (i, 0)),
        ]
        body = functools.partial(_head_body, blk=blk, with_base=True,
                                 nrows=nrows)
    else:
        out_shape = [jax.ShapeDtypeStruct((2, n, YWI), jnp.int32)]
        out_specs = [pl.BlockSpec((2, nrows, YWI), lambda i: (0, i, 0))]

        def body(x_ref, w1_ref, b1_ref, wagg_ref, bagg_ref, ma1_ref, ma2_ref,
                 brows_ref, y_ref):
            _head_body(x_ref, w1_ref, b1_ref, wagg_ref, bagg_ref, ma1_ref,
                       ma2_ref, brows_ref, y_ref, None, None,
                       blk=blk, with_base=False, nrows=nrows)

    return pl.pallas_call(
        body, grid=grid, in_specs=in_specs, out_specs=out_specs,
        out_shape=out_shape,
    )(x, w1, b1, wagg, bagg, ma1, ma2, brows)


def _sc_edges(ys, a1t, idxs, nbs, ncks):
    mesh = plsc.VectorSubcoreMesh(core_axis_name="c", subcore_axis_name="s")
    nph = len(ys)

    @functools.partial(
        pl.kernel, mesh=mesh,
        out_type=jax.ShapeDtypeStruct((2, OUTPAD, HD), jnp.float32),
        compiler_params=pltpu.CompilerParams(use_tc_tiling_on_sc=False),
        scratch_types=[
            pltpu.VMEM_SHARED((OUTPAD, HD), jnp.float32),
            pltpu.VMEM_SHARED((A1PAD, 16), jnp.float32),
            pltpu.VMEM((32, HD), jnp.float32),
            pltpu.VMEM((2, BLKCH, 2, C), jnp.int32),
            pltpu.VMEM((2, C, 16), jnp.float32),
            pltpu.VMEM((2, C, YWI), jnp.int32),
            pltpu.VMEM((2, C, HD), jnp.float32),
            pltpu.SemaphoreType.DMA,
            pltpu.SemaphoreType.DMA,
            pltpu.SemaphoreType.DMA,
            pltpu.SemaphoreType.DMA,
            pltpu.SemaphoreType.DMA,
        ],
    )
    def k(*refs):
        y_hbms = refs[:nph]
        a1_hbm = refs[nph]
        i_hbms = refs[nph + 1:2 * nph + 1]
        (out_hbm, acc_sp, a1_sp, zbuf, idxb, a1c, yc, vout, sem_i,
         sem_a0, sem_a1, sem_y0, sem_y1) = refs[2 * nph + 1:]
        cid = lax.axis_index("c")
        sid = lax.axis_index("s")
        sem_a = [sem_a0, sem_a1]
        sem_y = [sem_y0, sem_y1]

        pltpu.sync_copy(a1_hbm.at[pl.ds(sid * (A1PAD // NT), A1PAD // NT)],
                        a1_sp.at[pl.ds(sid * (A1PAD // NT), A1PAD // NT)])

        def zrow(i, _):
            for j in range(HD // 16):
                zbuf[i, pl.ds(j * 16, 16)] = jnp.zeros((16,), jnp.float32)
            return 0
        lax.fori_loop(0, 32, zrow, 0)

        def zcp(i, _):
            pltpu.sync_copy(zbuf, acc_sp.at[pl.ds(sid * 640 + i * 32, 32)])
            return 0
        lax.fori_loop(0, 20, zcp, 0)
        plsc.subcore_barrier()

        def phase(i_hbm, y_hbm, nblk, nck):
            def issue_gathers(c, slot):
                par = (c // BLKCH) % 2
                ent = c % BLKCH
                pltpu.async_copy(a1_sp.at[idxb.at[par, ent, 0]],
                                 a1c.at[slot], sem_a[slot])
                pltpu.async_copy(y_hbm.at[cid].at[idxb.at[par, ent, 1]],
                                 yc.at[slot], sem_y[slot])

            def wait_gathers(slot):
                pltpu.make_async_copy(a1_hbm.at[pl.ds(0, C)],
                                      a1c.at[slot], sem_a[slot]).wait()
                pltpu.make_async_copy(y_hbm.at[0, pl.ds(0, C)],
                                      yc.at[slot], sem_y[slot]).wait()

            def compute(slot):
                hidx = [jnp.full((16,), h, jnp.int32) for h in range(H)]

                def unpk(w):
                    lo = lax.bitcast_convert_type(
                        lax.shift_left(w, 16), jnp.float32)
                    hi = lax.bitcast_convert_type(
                        w & jnp.int32(-65536), jnp.float32)
                    return lo, hi

                def edge4(t, _):
                    for u in range(4):
                        e = t * 4 + u
                        av = a1c[slot, e, :]
                        wt = yc[slot, e, pl.ds(4 * 32, 16)]
                        ys, _ = unpk(wt)
                        att = 1.0 / (1.0 + jnp.exp(-(av + ys)))
                        acc = [None] * (HD // 16)
                        for h in range(H):
                            bh = jnp.take_along_axis(
                                att, hidx[h], axis=0,
                                mode="promise_in_bounds")
                            for g in range(2):
                                w = yc[slot, e, pl.ds(h * 32 + g * 16, 16)]
                                f_lo, f_hi = unpk(w)
                                acc[g] = (f_lo * bh if acc[g] is None
                                          else acc[g] + f_lo * bh)
                                acc[g + 2] = (f_hi * bh if acc[g + 2] is None
                                              else acc[g + 2] + f_hi * bh)
                        for j in range(HD // 16):
                            vout[slot, e, pl.ds(j * 16, 16)] = acc[j]
                    return 0
                lax.fori_loop(0, C // 4, edge4, 0)

            pltpu.sync_copy(i_hbm.at[sid, 0], idxb.at[0])
            issue_gathers(0, 0)

            def pair(p, _):
                for kk in range(2):
                    c = 2 * p + kk
                    if kk == 0:
                        @pl.when(jnp.logical_and(c % BLKCH == 0,
                                                 c // BLKCH + 1 < nblk))
                        def _():
                            nb = c // BLKCH + 1
                            pltpu.async_copy(i_hbm.at[sid, nb],
                                             idxb.at[nb % 2], sem_i)
                    @pl.when(c + 1 < nck)
                    def _():
                        @pl.when((c + 1) % BLKCH == 0)
                        def _():
                            pltpu.make_async_copy(
                                i_hbm.at[sid, 0], idxb.at[0], sem_i).wait()
                        issue_gathers(c + 1, (kk + 1) % 2)
                    wait_gathers(kk)
                    compute(kk)
                    par = (c // BLKCH) % 2
                    ent = c % BLKCH
                    pltpu.sync_copy(vout.at[kk],
                                    acc_sp.at[idxb.at[par, ent, 0]],
                                    add=True)
                return 0
            lax.fori_loop(0, nck // 2, pair, 0)

        for ph in range(nph):
            phase(i_hbms[ph], y_hbms[ph], nbs[ph], ncks[ph])
        plsc.subcore_barrier()

        def wcp(i, _):
            off = sid * 640 + i * 64
            pltpu.sync_copy(acc_sp.at[pl.ds(off, 64)],
                            out_hbm.at[cid, pl.ds(off, 64)])
            return 0
        lax.fori_loop(0, 10, wcp, 0)

    return k(*ys, a1t, *idxs)


def kernel(x0_1, x1, x2, adj0_row, adj0_col, adj1_row, adj1_col,
           adj2_row, adj2_col, W1, b1, a1w, a1b, a2w, a2b, Wagg, bagg):
    f32 = jnp.float32
    i32 = jnp.int32

    W1 = W1.astype(jnp.bfloat16)
    Wagg = Wagg.astype(jnp.bfloat16)
    ma1 = jnp.stack([jnp.zeros((D, 16), f32).at[:, h].set(a1w[h])
                     for h in range(H)])
    ma2 = jnp.stack([jnp.zeros((D, 16), f32).at[:, h].set(a2w[h])
                     for h in range(H)])
    brows = (jnp.zeros((8, 16), f32)
             .at[0, :H].set(a1b)
             .at[1, :H].set(a2b))

    y0, a1t, base = _dense_call(x0_1, W1, b1, Wagg, bagg, ma1, ma2, brows,
                                blk=1, with_base=True)
    (y2,) = _dense_call(x2, W1, b1, Wagg, bagg, ma1, ma2, brows,
                        blk=3, with_base=False, nrows=2000)

    a1p = jnp.concatenate([a1t, jnp.zeros((A1PAD - N0, 16), f32)])

    def pack_edges(rows, cols, nblk, nck):
        e = rows.shape[0]
        ep = NT * C * nck
        slots = nblk * BLKCH
        rp = jnp.concatenate([rows, jnp.full((ep - e,), PADROW, i32)])
        cp = jnp.concatenate([cols, jnp.zeros((ep - e,), i32)])
        r3 = rp.reshape(NT, nck, C)
        c3 = cp.reshape(NT, nck, C)
        r3 = jnp.pad(r3, ((0, 0), (0, slots - nck), (0, 0)),
                     constant_values=PADROW)
        c3 = jnp.pad(c3, ((0, 0), (0, slots - nck), (0, 0)))
        rc = jnp.stack([r3.reshape(NT, nblk, BLKCH, C),
                        c3.reshape(NT, nblk, BLKCH, C)], axis=3)
        return rc

    i0 = pack_edges(adj0_row, adj0_col, NB0, NCK0)
    i1 = pack_edges(adj1_row, adj1_col, NB1, NCK1)
    i2 = pack_edges(adj2_row, adj2_col, NB2, NCK2)

    outa = _sc_edges([y0, y2], a1p, [i0, i2], [NB0, NB2], [NCK0, NCK2])

    (y1,) = _dense_call(x1, W1, b1, Wagg, bagg, ma1, ma2, brows,
                        blk=2, with_base=False, nrows=2000)
    outb = _sc_edges([y1], a1p, [i1], [NB1], [NCK1])

    outp = outa + outb
    return base + jnp.concatenate([outp[0, :N0], outp[1, :N0]], axis=1)
